# Initial kernel scaffold; baseline (speedup 1.0000x reference)
import jax
import jax.numpy as jnp
from jax.experimental import pallas as pl


def kernel(h, adj_list, labels, W1, b1, W2, b2):
    raise NotImplementedError("write your pallas kernel here")


# speedup vs baseline: 1.0894x; 1.0894x over previous
import functools

import numpy as np
import jax
import jax.numpy as jnp
from jax import lax
from jax.experimental import pallas as pl
from jax.experimental.pallas import tpu as pltpu
from jax.experimental.pallas import tpu_sc as plsc

N = 10000
R = 2
E = 160000
F_IN = 256
H = 128
C = 16
ALPHA = 0.1
LAMDA2 = 0.01
LAMDA1 = 1.0 / ALPHA - 1.0
LAYER_NUM = 2
ITERS = 15

NC = 2
NS = 16
NW = NC * NS
BLK = 128
NBLK = 40
EPW = NBLK * BLK
EPAD = NW * EPW
NP = 10240
ROWS_PER_TILE = NP // NS

_L16 = 16


def _rsqrt_sc(x):
    i = plsc.bitcast(x, jnp.int32)
    y = plsc.bitcast(jnp.int32(0x5F3759DF) - (i >> 1), jnp.float32)
    for _ in range(3):
        y = y * (1.5 - 0.5 * x * y * y)
    return y


def _tanh_pos_sc(y):
    t = jnp.exp(-2.0 * y)
    return (1.0 - t) / (1.0 + t)


def _deg_body(row0_hbm, mask0_hbm, row1_hbm, mask1_hbm, deg0_out, deg1_out,
              deg_v, ridx_v, mask_v):
    cid = lax.axis_index("c")
    sid = lax.axis_index("s")
    wid = cid * NS + sid
    base = wid * EPW

    zeros16 = jnp.zeros((_L16,), jnp.float32)
    izero16 = jnp.zeros((_L16,), jnp.int32)
    for r, (row_hbm, mask_hbm, deg_out) in enumerate(
            ((row0_hbm, mask0_hbm, deg0_out), (row1_hbm, mask1_hbm, deg1_out))):
        def zero_body(i, _):
            deg_v[0, pl.ds(i * _L16, _L16)] = zeros16
            return 0
        lax.fori_loop(0, N // _L16, zero_body, 0)

        def blk_body(b, _):
            off = base + b * BLK
            pltpu.sync_copy(row_hbm.at[pl.ds(off, BLK)], ridx_v)
            pltpu.sync_copy(mask_hbm.at[pl.ds(off, BLK)], mask_v)
            for g in range(BLK // _L16):
                r16 = ridx_v[pl.ds(g * _L16, _L16)]
                m16 = mask_v[pl.ds(g * _L16, _L16)]
                plsc.addupdate_scatter(deg_v, [izero16, r16], m16)
            return 0
        lax.fori_loop(0, NBLK, blk_body, 0)
        pltpu.sync_copy(deg_v, deg_out.at[wid])


def _deg_kernel(row0, mask0, row1, mask1):
    mesh = plsc.VectorSubcoreMesh(core_axis_name="c", subcore_axis_name="s")
    return pl.kernel(
        _deg_body,
        out_type=(
            jax.ShapeDtypeStruct((NW, 1, N), jnp.float32),
            jax.ShapeDtypeStruct((NW, 1, N), jnp.float32),
        ),
        mesh=mesh,
        scratch_types=[
            pltpu.VMEM((1, N), jnp.float32),
            pltpu.VMEM((BLK,), jnp.int32),
            pltpu.VMEM((BLK,), jnp.float32),
        ],
        compiler_params=pltpu.CompilerParams(needs_layout_passes=False),
    )(row0, mask0, row1, mask1)


def _edge_body(h1_hbm, ds_hbm, row_hbm, col_hbm, mask_hbm,
               zeros_hbm, fout_hbm, s_hbm,
               accum, ds_v, ridx_v, cidx_v, mask_v,
               hr_v, hc_v, s_v, sem_r, sem_c):
    cid = lax.axis_index("c")
    sid = lax.axis_index("s")
    wid = cid * NS + sid
    base = wid * EPW

    @pl.when(sid == 0)
    def _():
        pltpu.sync_copy(zeros_hbm, accum)

    pltpu.sync_copy(ds_hbm, ds_v)
    plsc.subcore_barrier()

    def blk_body(b, carry):
        s_num, s_den = carry
        off = base + b * BLK
        pltpu.sync_copy(row_hbm.at[pl.ds(off, BLK)], ridx_v)
        pltpu.sync_copy(col_hbm.at[pl.ds(off, BLK)], cidx_v)
        pltpu.sync_copy(mask_hbm.at[pl.ds(off, BLK)], mask_v)
        cp_r = pltpu.async_copy(h1_hbm.at[ridx_v], hr_v, sem_r)
        cp_c = pltpu.async_copy(h1_hbm.at[cidx_v], hc_v, sem_c)
        cp_r.wait()
        cp_c.wait()

        iota16 = lax.iota(jnp.int32, _L16)
        e_base = [iota16 + g * _L16 for g in range(BLK // _L16)]

        def f_body(f, accs):
            fv = jnp.full((_L16,), f, dtype=jnp.int32)
            out = []
            for (d, qa, qb), e16 in zip(accs, e_base):
                a = plsc.load_gather(hr_v, [e16, fv])
                b = plsc.load_gather(hc_v, [e16, fv])
                out.append((d + a * b, qa + a * a, qb + b * b))
            return tuple(out)

        z16 = jnp.zeros((_L16,), jnp.float32)
        dots = lax.fori_loop(
            0, H, f_body,
            tuple((z16, z16, z16) for _ in range(BLK // _L16)),
            unroll=4)

        norms = []
        for g in range(BLK // _L16):
            sl = pl.ds(g * _L16, _L16)
            r16 = ridx_v[sl]
            c16 = cidx_v[sl]
            m16 = mask_v[sl]
            dot16, qr, qc = dots[g]
            sr = plsc.load_gather(ds_v, [r16])
            sc = plsc.load_gather(ds_v, [c16])
            d_raw = jnp.maximum(qr + qc - 2.0 * dot16, 0.0)
            d_scl = jnp.maximum(sr * sr * qr + sc * sc * qc
                                - 2.0 * (sr * sc) * dot16, 0.0)
            g_raw = _tanh_pos_sc(_rsqrt_sc(d_raw + 1e-4))
            g_scl = _tanh_pos_sc(_rsqrt_sc(d_scl + 1e-4))
            srsc = sr * sc
            norms.append(g_scl * srsc * srsc * m16)
            s_num = s_num + g_raw * m16
            s_den = s_den + m16

        def s_body(f, _):
            fv = jnp.full((_L16,), f, dtype=jnp.int32)
            for g in range(BLK // _L16):
                val = plsc.load_gather(hr_v, [e_base[g], fv]) * norms[g]
                plsc.store_scatter(hr_v, [e_base[g], fv], val)
            return 0
        lax.fori_loop(0, H, s_body, 0, unroll=4)

        pltpu.sync_copy(hr_v, accum.at[cidx_v], add=True)
        return (s_num, s_den)

    init = (jnp.zeros((_L16,), jnp.float32), jnp.zeros((_L16,), jnp.float32))
    s_num, s_den = lax.fori_loop(0, NBLK, blk_body, init)

    lanes = lax.iota(jnp.int32, _L16)
    svec = jnp.where(lanes == 0, jnp.sum(s_num),
                     jnp.where(lanes == 1, jnp.sum(s_den), 0.0))
    s_v[0, :] = svec
    pltpu.sync_copy(s_v, s_hbm.at[wid])

    plsc.subcore_barrier()
    pltpu.sync_copy(accum.at[pl.ds(sid * ROWS_PER_TILE, ROWS_PER_TILE)],
                    fout_hbm.at[cid, pl.ds(sid * ROWS_PER_TILE, ROWS_PER_TILE)])


def _edge_kernel(h1, ds_r, row_r, col_r, mask_r, zeros):
    mesh = plsc.VectorSubcoreMesh(core_axis_name="c", subcore_axis_name="s")
    return pl.kernel(
        _edge_body,
        out_type=(
            jax.ShapeDtypeStruct((NC, NP, H), jnp.float32),
            jax.ShapeDtypeStruct((NW, 1, _L16), jnp.float32),
        ),
        mesh=mesh,
        scratch_types=[
            pltpu.VMEM_SHARED((NP, H), jnp.float32),
            pltpu.VMEM((N,), jnp.float32),
            pltpu.VMEM((BLK,), jnp.int32),
            pltpu.VMEM((BLK,), jnp.int32),
            pltpu.VMEM((BLK,), jnp.float32),
            pltpu.VMEM((BLK, H), jnp.float32),
            pltpu.VMEM((BLK, H), jnp.float32),
            pltpu.VMEM((1, _L16), jnp.float32),
            pltpu.SemaphoreType.DMA,
            pltpu.SemaphoreType.DMA,
        ],
        compiler_params=pltpu.CompilerParams(needs_layout_passes=False),
    )(h1, ds_r, row_r, col_r, mask_r, zeros)


MB = 1000
MG = N // MB


def _mm1_body(h_ref, w_ref, b_ref, h1_ref):
    h1 = jnp.dot(h_ref[...], w_ref[...], preferred_element_type=jnp.float32)
    h1_ref[...] = jnp.maximum(h1 + b_ref[...], 0.0)


def _mm1(h, w1t, b1):
    return pl.pallas_call(
        _mm1_body,
        grid=(MG,),
        in_specs=[
            pl.BlockSpec((MB, F_IN), lambda i: (i, 0)),
            pl.BlockSpec((F_IN, H), lambda i: (0, 0)),
            pl.BlockSpec((1, H), lambda i: (0, 0)),
        ],
        out_specs=pl.BlockSpec((MB, H), lambda i: (i, 0)),
        out_shape=jax.ShapeDtypeStruct((N, H), jnp.float32),
    )(h, w1t, b1)


def _prep_body(p0_ref, p1_ref, ds_ref):
    d0 = jnp.sum(p0_ref[...], axis=(0, 1))
    d1 = jnp.sum(p1_ref[...], axis=(0, 1))
    deg = jnp.maximum(jnp.stack([d0, d1]), 1.0)
    ds_ref[...] = lax.sqrt(lax.rsqrt(deg))


def _prep(deg0, deg1):
    return pl.pallas_call(
        _prep_body,
        out_shape=jax.ShapeDtypeStruct((R, N), jnp.float32),
    )(deg0, deg1)


def _combine_body(raw_ref, f0_ref, f1_ref, s0_ref, s1_ref, h1_ref):
    s0 = s0_ref[...]
    s1 = s1_ref[...]
    sr0 = jnp.sum(s0[:, 0, 0]) / jnp.sum(s0[:, 0, 1])
    sr1 = jnp.sum(s1[:, 0, 0]) / jnp.sum(s1[:, 0, 1])
    l1tr = jnp.abs(sr0) + jnp.abs(sr1)
    fi = l1tr + 2.0 * LAMDA2 / LAMDA1
    cl = 2.0 * LAMDA2 / LAMDA1
    u0 = jnp.float32(0.5)
    u1 = jnp.float32(0.5)
    for it in range(ITERS):
        t_t = np.float32(np.sqrt(2.0 * np.log(R) / (it + 1.0))) / fi
        u0t = u0 * jnp.exp(-t_t * (cl * u0 + sr0))
        u1t = u1 * jnp.exp(-t_t * (cl * u1 + sr1))
        ssum = u0t + u1t
        u0 = u0t / ssum
        u1 = u1t / ssum
    f_sum = u0 * (f0_ref[0] + f0_ref[1]) + u1 * (f1_ref[0] + f1_ref[1])
    h1_ref[...] = ALPHA * raw_ref[...] + (1.0 - ALPHA) * f_sum


def _combine(raw, f0, f1, s0, s1):
    return pl.pallas_call(
        _combine_body,
        grid=(MG,),
        in_specs=[
            pl.BlockSpec((MB, H), lambda i: (i, 0)),
            pl.BlockSpec((NC, MB, H), lambda i: (0, i, 0)),
            pl.BlockSpec((NC, MB, H), lambda i: (0, i, 0)),
            pl.BlockSpec((NW, 1, _L16), lambda i: (0, 0, 0)),
            pl.BlockSpec((NW, 1, _L16), lambda i: (0, 0, 0)),
        ],
        out_specs=pl.BlockSpec((MB, H), lambda i: (i, 0)),
        out_shape=jax.ShapeDtypeStruct((N, H), jnp.float32),
    )(raw, f0, f1, s0, s1)


def _final_body(h1_ref, w_ref, b_ref, lp_ref, lg_ref):
    logits = jnp.dot(h1_ref[...], w_ref[...], preferred_element_type=jnp.float32)
    logits = logits + b_ref[...]
    m = jnp.max(logits, axis=1, keepdims=True)
    lse = m + jnp.log(jnp.sum(jnp.exp(logits - m), axis=1, keepdims=True))
    lg_ref[...] = logits
    lp_ref[...] = logits - lse


def _final(h1, w2t, b2):
    return pl.pallas_call(
        _final_body,
        grid=(MG,),
        in_specs=[
            pl.BlockSpec((MB, H), lambda i: (i, 0)),
            pl.BlockSpec((H, C), lambda i: (0, 0)),
            pl.BlockSpec((1, C), lambda i: (0, 0)),
        ],
        out_specs=(
            pl.BlockSpec((MB, C), lambda i: (i, 0)),
            pl.BlockSpec((MB, C), lambda i: (i, 0)),
        ),
        out_shape=(
            jax.ShapeDtypeStruct((N, C), jnp.float32),
            jax.ShapeDtypeStruct((N, C), jnp.float32),
        ),
    )(h1, w2t, b2)


def kernel(h, adj_list, labels, W1, b1, W2, b2):
    rows, cols, masks = [], [], []
    arange = jnp.arange(EPAD, dtype=jnp.int32)
    for r in range(R):
        rr, cc = jnp.nonzero(adj_list[r], size=EPAD, fill_value=0)
        cnt = jnp.count_nonzero(adj_list[r])
        rows.append(rr.astype(jnp.int32))
        cols.append(cc.astype(jnp.int32))
        masks.append((arange < cnt).astype(jnp.float32))

    h1 = _mm1(h, W1.T, b1.reshape(1, H))
    raw = h1
    deg0, deg1 = _deg_kernel(rows[0], masks[0], rows[1], masks[1])
    ds = _prep(deg0, deg1)
    zeros = jnp.zeros((NP, H), jnp.float32)

    for _ in range(LAYER_NUM):
        f_list, s_list = [], []
        for r in range(R):
            fpart, spart = _edge_kernel(h1, ds[r],
                                        rows[r], cols[r], masks[r], zeros)
            f_list.append(fpart)
            s_list.append(spart)
        h1 = _combine(raw, f_list[0], f_list[1], s_list[0], s_list[1])

    return _final(h1, W2.T, b2.reshape(1, C))


# speedup vs baseline: 4.1784x; 3.8354x over previous
import functools

import numpy as np
import jax
import jax.numpy as jnp
from jax import lax
from jax.experimental import pallas as pl
from jax.experimental.pallas import tpu as pltpu
from jax.experimental.pallas import tpu_sc as plsc

N = 10000
R = 2
E = 160000
F_IN = 256
H = 128
C = 16
ALPHA = 0.1
LAMDA2 = 0.01
LAMDA1 = 1.0 / ALPHA - 1.0
LAYER_NUM = 2
ITERS = 15

NC = 2
NS = 16
NW = NC * NS
BLK = 128
RPW = 320
CAPW = 160128
NGRP = 625
NP = 10240
ROWS_PER_TILE = NP // NS

_L16 = 16


def _rsqrt_sc(x):
    i = plsc.bitcast(x, jnp.int32)
    y = plsc.bitcast(jnp.int32(0x5F3759DF) - (i >> 1), jnp.float32)
    for _ in range(3):
        y = y * (1.5 - 0.5 * x * y * y)
    return y


def _tanh_pos_sc(y):
    t = jnp.exp(-2.0 * y)
    return (1.0 - t) / (1.0 + t)


def _ext_body(adj_hbm, r0_out, c0_out, n0_out, r1_out, c1_out, n1_out,
              abuf, rbuf, cbuf, cv):
    cid = lax.axis_index("c")
    sid = lax.axis_index("s")
    wid = cid * NS + sid
    row0 = wid * RPW
    nrows = jnp.minimum(RPW, N - row0)
    ngrp = nrows // 8

    iota16 = lax.iota(jnp.int32, _L16)
    izero16 = jnp.zeros((_L16,), jnp.int32)

    def z_body(i, _):
        rbuf[pl.ds(i * _L16, _L16)] = izero16
        cbuf[pl.ds(i * _L16, _L16)] = izero16
        return 0
    lax.fori_loop(0, (NGRP * _L16 + BLK) // _L16, z_body, 0)

    for r, (r_out, c_out, n_out) in enumerate(
            ((r0_out, c0_out, n0_out), (r1_out, c1_out, n1_out))):
        def grp_body(grp, carry):
            ptrv, hptr = carry
            gr0 = pl.multiple_of(row0 + grp * 8, 8)
            pltpu.sync_copy(adj_hbm.at[r, pl.ds(gr0, 8), :], abuf)
            for lr in range(8):
                rowvec = izero16 + (gr0 + lr)

                def g_body(g, ptrv):
                    vals = abuf[lr, pl.ds(g * _L16, _L16)]
                    m = vals != 0.0
                    mi = jnp.where(m, 1, 0).astype(jnp.int32)
                    pos = ptrv + plsc.cumsum(mi) - mi
                    colv = iota16 + g * _L16
                    plsc.store_scatter(rbuf, [pos], rowvec, mask=m)
                    plsc.store_scatter(cbuf, [pos], colv, mask=m)
                    return ptrv + plsc.all_reduce_population_count(m)

                ptrv = lax.fori_loop(0, NGRP, g_body, ptrv, unroll=2)

                ptr = jnp.max(ptrv)
                nfull = lax.div(ptr, BLK)

                def fl_body(k, _):
                    off = pl.multiple_of(k * BLK, BLK)
                    dst = pl.multiple_of(hptr + k * BLK, BLK)
                    pltpu.sync_copy(rbuf.at[pl.ds(off, BLK)],
                                    r_out.at[wid, 0, pl.ds(dst, BLK)])
                    pltpu.sync_copy(cbuf.at[pl.ds(off, BLK)],
                                    c_out.at[wid, 0, pl.ds(dst, BLK)])
                    return 0
                lax.fori_loop(0, nfull, fl_body, 0)

                rem = ptr - nfull * BLK
                srcb = pl.multiple_of(nfull * BLK, BLK)
                for t in range(BLK // _L16):
                    rv = rbuf[pl.ds(srcb + t * _L16, _L16)]
                    cvv = cbuf[pl.ds(srcb + t * _L16, _L16)]
                    rbuf[pl.ds(t * _L16, _L16)] = rv
                    cbuf[pl.ds(t * _L16, _L16)] = cvv
                hptr = hptr + nfull * BLK
                ptrv = izero16 + rem
            return (ptrv, hptr)

        ptrv, hptr = lax.fori_loop(0, ngrp, grp_body,
                                   (izero16, jnp.int32(0)))
        ptr = jnp.max(ptrv)
        dst = pl.multiple_of(hptr, BLK)
        pltpu.sync_copy(rbuf.at[pl.ds(0, BLK)], r_out.at[wid, 0, pl.ds(dst, BLK)])
        pltpu.sync_copy(cbuf.at[pl.ds(0, BLK)], c_out.at[wid, 0, pl.ds(dst, BLK)])
        cv[0, :] = jnp.where(iota16 == 0, hptr + ptr, 0)
        pltpu.sync_copy(cv, n_out.at[wid])


def _extract(adj_list):
    mesh = plsc.VectorSubcoreMesh(core_axis_name="c", subcore_axis_name="s")
    slab = jax.ShapeDtypeStruct((NW, 1, CAPW), jnp.int32)
    cnts = jax.ShapeDtypeStruct((NW, 1, _L16), jnp.int32)
    return pl.kernel(
        _ext_body,
        out_type=(slab, slab, cnts, slab, slab, cnts),
        mesh=mesh,
        scratch_types=[
            pltpu.VMEM((8, N), jnp.float32),
            pltpu.VMEM((NGRP * _L16 + BLK,), jnp.int32),
            pltpu.VMEM((NGRP * _L16 + BLK,), jnp.int32),
            pltpu.VMEM((1, _L16), jnp.int32),
        ],
        compiler_params=pltpu.CompilerParams(needs_layout_passes=False),
    )(adj_list)


def _deg_body(row0_hbm, n0_hbm, row1_hbm, n1_hbm, deg0_out, deg1_out,
              deg_v, ridx_v, cnt_v):
    cid = lax.axis_index("c")
    sid = lax.axis_index("s")
    wid = cid * NS + sid

    iota16 = lax.iota(jnp.int32, _L16)
    zeros16 = jnp.zeros((_L16,), jnp.float32)
    izero16 = jnp.zeros((_L16,), jnp.int32)
    for r, (row_hbm, n_hbm, deg_out) in enumerate(
            ((row0_hbm, n0_hbm, deg0_out), (row1_hbm, n1_hbm, deg1_out))):
        def zero_body(i, _):
            deg_v[0, pl.ds(i * _L16, _L16)] = zeros16
            return 0
        lax.fori_loop(0, N // _L16, zero_body, 0)

        pltpu.sync_copy(n_hbm.at[wid], cnt_v)
        cnt_w = jnp.max(cnt_v[0, :])
        nblk = lax.div(cnt_w + BLK - 1, BLK)

        def blk_body(b, _):
            off = pl.multiple_of(b * BLK, BLK)
            pltpu.sync_copy(row_hbm.at[wid, 0, pl.ds(off, BLK)], ridx_v)
            for g in range(BLK // _L16):
                r16 = ridx_v[pl.ds(g * _L16, _L16)]
                idx16 = iota16 + (b * BLK + g * _L16)
                m16 = jnp.where(idx16 < cnt_w, 1.0, 0.0)
                plsc.addupdate_scatter(deg_v, [izero16, r16], m16)
            return 0
        lax.fori_loop(0, nblk, blk_body, 0)
        pltpu.sync_copy(deg_v, deg_out.at[wid])


def _deg_kernel(row0, n0, row1, n1):
    mesh = plsc.VectorSubcoreMesh(core_axis_name="c", subcore_axis_name="s")
    return pl.kernel(
        _deg_body,
        out_type=(
            jax.ShapeDtypeStruct((NW, 1, N), jnp.float32),
            jax.ShapeDtypeStruct((NW, 1, N), jnp.float32),
        ),
        mesh=mesh,
        scratch_types=[
            pltpu.VMEM((1, N), jnp.float32),
            pltpu.VMEM((BLK,), jnp.int32),
            pltpu.VMEM((1, _L16), jnp.int32),
        ],
        compiler_params=pltpu.CompilerParams(needs_layout_passes=False),
    )(row0, n0, row1, n1)


def _edge_body(h1_hbm, ds_hbm, row_hbm, col_hbm, n_hbm,
               zeros_hbm, fout_hbm, s_hbm,
               accum, ds_v, ridx_v, cidx_v, cnt_v,
               hr_v, hc_v, s_v, sem_r, sem_c):
    cid = lax.axis_index("c")
    sid = lax.axis_index("s")
    wid = cid * NS + sid

    @pl.when(sid == 0)
    def _():
        pltpu.sync_copy(zeros_hbm, accum)

    pltpu.sync_copy(ds_hbm, ds_v)
    pltpu.sync_copy(n_hbm.at[wid], cnt_v)
    cnt_w = jnp.max(cnt_v[0, :])
    nblk = lax.div(cnt_w + BLK - 1, BLK)
    plsc.subcore_barrier()

    def blk_body(b, carry):
        s_num, s_den = carry
        off = pl.multiple_of(b * BLK, BLK)
        pltpu.sync_copy(row_hbm.at[wid, 0, pl.ds(off, BLK)], ridx_v)
        pltpu.sync_copy(col_hbm.at[wid, 0, pl.ds(off, BLK)], cidx_v)
        cp_r = pltpu.async_copy(h1_hbm.at[ridx_v], hr_v, sem_r)
        cp_c = pltpu.async_copy(h1_hbm.at[cidx_v], hc_v, sem_c)
        cp_r.wait()
        cp_c.wait()

        iota16 = lax.iota(jnp.int32, _L16)
        e_base = [iota16 + g * _L16 for g in range(BLK // _L16)]

        def f_body(f, accs):
            fv = jnp.full((_L16,), f, dtype=jnp.int32)
            out = []
            for (d, qa, qb), e16 in zip(accs, e_base):
                a = plsc.load_gather(hr_v, [e16, fv])
                b = plsc.load_gather(hc_v, [e16, fv])
                out.append((d + a * b, qa + a * a, qb + b * b))
            return tuple(out)

        z16 = jnp.zeros((_L16,), jnp.float32)
        dots = lax.fori_loop(
            0, H, f_body,
            tuple((z16, z16, z16) for _ in range(BLK // _L16)),
            unroll=4)

        norms = []
        for g in range(BLK // _L16):
            sl = pl.ds(g * _L16, _L16)
            r16 = ridx_v[sl]
            c16 = cidx_v[sl]
            idx16 = iota16 + (b * BLK + g * _L16)
            m16 = jnp.where(idx16 < cnt_w, 1.0, 0.0)
            dot16, qr, qc = dots[g]
            sr = plsc.load_gather(ds_v, [r16])
            sc = plsc.load_gather(ds_v, [c16])
            d_raw = jnp.maximum(qr + qc - 2.0 * dot16, 0.0)
            d_scl = jnp.maximum(sr * sr * qr + sc * sc * qc
                                - 2.0 * (sr * sc) * dot16, 0.0)
            g_raw = _tanh_pos_sc(_rsqrt_sc(d_raw + 1e-4))
            g_scl = _tanh_pos_sc(_rsqrt_sc(d_scl + 1e-4))
            srsc = sr * sc
            norms.append(g_scl * srsc * srsc * m16)
            s_num = s_num + g_raw * m16
            s_den = s_den + m16

        def s_body(f, _):
            fv = jnp.full((_L16,), f, dtype=jnp.int32)
            for g in range(BLK // _L16):
                val = plsc.load_gather(hr_v, [e_base[g], fv]) * norms[g]
                plsc.store_scatter(hr_v, [e_base[g], fv], val)
            return 0
        lax.fori_loop(0, H, s_body, 0, unroll=4)

        pltpu.sync_copy(hr_v, accum.at[cidx_v], add=True)
        return (s_num, s_den)

    init = (jnp.zeros((_L16,), jnp.float32), jnp.zeros((_L16,), jnp.float32))
    s_num, s_den = lax.fori_loop(0, nblk, blk_body, init)

    lanes = lax.iota(jnp.int32, _L16)
    svec = jnp.where(lanes == 0, jnp.sum(s_num),
                     jnp.where(lanes == 1, jnp.sum(s_den), 0.0))
    s_v[0, :] = svec
    pltpu.sync_copy(s_v, s_hbm.at[wid])

    plsc.subcore_barrier()
    pltpu.sync_copy(accum.at[pl.ds(sid * ROWS_PER_TILE, ROWS_PER_TILE)],
                    fout_hbm.at[cid, pl.ds(sid * ROWS_PER_TILE, ROWS_PER_TILE)])


def _edge_kernel(h1, ds_r, row_r, col_r, n_r, zeros):
    mesh = plsc.VectorSubcoreMesh(core_axis_name="c", subcore_axis_name="s")
    return pl.kernel(
        _edge_body,
        out_type=(
            jax.ShapeDtypeStruct((NC, NP, H), jnp.float32),
            jax.ShapeDtypeStruct((NW, 1, _L16), jnp.float32),
        ),
        mesh=mesh,
        scratch_types=[
            pltpu.VMEM_SHARED((NP, H), jnp.float32),
            pltpu.VMEM((N,), jnp.float32),
            pltpu.VMEM((BLK,), jnp.int32),
            pltpu.VMEM((BLK,), jnp.int32),
            pltpu.VMEM((1, _L16), jnp.int32),
            pltpu.VMEM((BLK, H), jnp.float32),
            pltpu.VMEM((BLK, H), jnp.float32),
            pltpu.VMEM((1, _L16), jnp.float32),
            pltpu.SemaphoreType.DMA,
            pltpu.SemaphoreType.DMA,
        ],
        compiler_params=pltpu.CompilerParams(needs_layout_passes=False),
    )(h1, ds_r, row_r, col_r, n_r, zeros)


MB = 1000
MG = N // MB


def _mm1_body(h_ref, w_ref, b_ref, h1_ref):
    h1 = jnp.dot(h_ref[...], w_ref[...], preferred_element_type=jnp.float32)
    h1_ref[...] = jnp.maximum(h1 + b_ref[...], 0.0)


def _mm1(h, w1t, b1):
    return pl.pallas_call(
        _mm1_body,
        grid=(MG,),
        in_specs=[
            pl.BlockSpec((MB, F_IN), lambda i: (i, 0)),
            pl.BlockSpec((F_IN, H), lambda i: (0, 0)),
            pl.BlockSpec((1, H), lambda i: (0, 0)),
        ],
        out_specs=pl.BlockSpec((MB, H), lambda i: (i, 0)),
        out_shape=jax.ShapeDtypeStruct((N, H), jnp.float32),
    )(h, w1t, b1)


def _prep_body(p0_ref, p1_ref, ds_ref):
    d0 = jnp.sum(p0_ref[...], axis=(0, 1))
    d1 = jnp.sum(p1_ref[...], axis=(0, 1))
    deg = jnp.maximum(jnp.stack([d0, d1]), 1.0)
    ds_ref[...] = lax.sqrt(lax.rsqrt(deg))


def _prep(deg0, deg1):
    return pl.pallas_call(
        _prep_body,
        out_shape=jax.ShapeDtypeStruct((R, N), jnp.float32),
    )(deg0, deg1)


def _combine_body(raw_ref, f0_ref, f1_ref, s0_ref, s1_ref, h1_ref):
    s0 = s0_ref[...]
    s1 = s1_ref[...]
    sr0 = jnp.sum(s0[:, 0, 0]) / jnp.sum(s0[:, 0, 1])
    sr1 = jnp.sum(s1[:, 0, 0]) / jnp.sum(s1[:, 0, 1])
    l1tr = jnp.abs(sr0) + jnp.abs(sr1)
    fi = l1tr + 2.0 * LAMDA2 / LAMDA1
    cl = 2.0 * LAMDA2 / LAMDA1
    u0 = jnp.float32(0.5)
    u1 = jnp.float32(0.5)
    for it in range(ITERS):
        t_t = np.float32(np.sqrt(2.0 * np.log(R) / (it + 1.0))) / fi
        u0t = u0 * jnp.exp(-t_t * (cl * u0 + sr0))
        u1t = u1 * jnp.exp(-t_t * (cl * u1 + sr1))
        ssum = u0t + u1t
        u0 = u0t / ssum
        u1 = u1t / ssum
    f_sum = u0 * (f0_ref[0] + f0_ref[1]) + u1 * (f1_ref[0] + f1_ref[1])
    h1_ref[...] = ALPHA * raw_ref[...] + (1.0 - ALPHA) * f_sum


def _combine(raw, f0, f1, s0, s1):
    return pl.pallas_call(
        _combine_body,
        grid=(MG,),
        in_specs=[
            pl.BlockSpec((MB, H), lambda i: (i, 0)),
            pl.BlockSpec((NC, MB, H), lambda i: (0, i, 0)),
            pl.BlockSpec((NC, MB, H), lambda i: (0, i, 0)),
            pl.BlockSpec((NW, 1, _L16), lambda i: (0, 0, 0)),
            pl.BlockSpec((NW, 1, _L16), lambda i: (0, 0, 0)),
        ],
        out_specs=pl.BlockSpec((MB, H), lambda i: (i, 0)),
        out_shape=jax.ShapeDtypeStruct((N, H), jnp.float32),
    )(raw, f0, f1, s0, s1)


def _final_body(h1_ref, w_ref, b_ref, lp_ref, lg_ref):
    logits = jnp.dot(h1_ref[...], w_ref[...], preferred_element_type=jnp.float32)
    logits = logits + b_ref[...]
    m = jnp.max(logits, axis=1, keepdims=True)
    lse = m + jnp.log(jnp.sum(jnp.exp(logits - m), axis=1, keepdims=True))
    lg_ref[...] = logits
    lp_ref[...] = logits - lse


def _final(h1, w2t, b2):
    return pl.pallas_call(
        _final_body,
        grid=(MG,),
        in_specs=[
            pl.BlockSpec((MB, H), lambda i: (i, 0)),
            pl.BlockSpec((H, C), lambda i: (0, 0)),
            pl.BlockSpec((1, C), lambda i: (0, 0)),
        ],
        out_specs=(
            pl.BlockSpec((MB, C), lambda i: (i, 0)),
            pl.BlockSpec((MB, C), lambda i: (i, 0)),
        ),
        out_shape=(
            jax.ShapeDtypeStruct((N, C), jnp.float32),
            jax.ShapeDtypeStruct((N, C), jnp.float32),
        ),
    )(h1, w2t, b2)


def kernel(h, adj_list, labels, W1, b1, W2, b2):
    r0, c0, n0, r1, c1, n1 = _extract(adj_list)
    rows, cols, cnts = (r0, r1), (c0, c1), (n0, n1)

    h1 = _mm1(h, W1.T, b1.reshape(1, H))
    raw = h1
    deg0, deg1 = _deg_kernel(rows[0], cnts[0], rows[1], cnts[1])
    ds = _prep(deg0, deg1)
    zeros = jnp.zeros((NP, H), jnp.float32)

    for _ in range(LAYER_NUM):
        f_list, s_list = [], []
        for r in range(R):
            fpart, spart = _edge_kernel(h1, ds[r],
                                        rows[r], cols[r], cnts[r], zeros)
            f_list.append(fpart)
            s_list.append(spart)
        h1 = _combine(raw, f_list[0], f_list[1], s_list[0], s_list[1])

    return _final(h1, W2.T, b2.reshape(1, C))


# speedup vs baseline: 4.4423x; 1.0631x over previous
import functools

import numpy as np
import jax
import jax.numpy as jnp
from jax import lax
from jax.experimental import pallas as pl
from jax.experimental.pallas import tpu as pltpu
from jax.experimental.pallas import tpu_sc as plsc

N = 10000
R = 2
E = 160000
F_IN = 256
H = 128
C = 16
ALPHA = 0.1
LAMDA2 = 0.01
LAMDA1 = 1.0 / ALPHA - 1.0
LAYER_NUM = 2
ITERS = 15

NC = 2
NS = 16
NW = NC * NS
BLK = 128
SUB = 64
NSB = 32
RPT = 640
RPW = 320
CAPW = 161792
NGRP = 625

_L16 = 16


def _rsqrt_sc(x):
    i = plsc.bitcast(x, jnp.int32)
    y = plsc.bitcast(jnp.int32(0x5F3759DF) - (i >> 1), jnp.float32)
    for _ in range(3):
        y = y * (1.5 - 0.5 * x * y * y)
    return y


def _tanh_pos_sc(y):
    t = jnp.exp(-2.0 * y)
    return (1.0 - t) / (1.0 + t)


def _ext_body(adj_hbm, r0_out, c0_out, n0_out, r1_out, c1_out, n1_out,
              abuf, rbuf, cbuf, cv):
    cid = lax.axis_index("c")
    sid = lax.axis_index("s")
    wid = cid * NS + sid
    row0 = wid * RPW
    nrows = jnp.minimum(RPW, N - row0)
    ngrp = nrows // 8

    iota16 = lax.iota(jnp.int32, _L16)
    izero16 = jnp.zeros((_L16,), jnp.int32)

    def z_body(i, _):
        rbuf[pl.ds(i * _L16, _L16)] = izero16
        cbuf[pl.ds(i * _L16, _L16)] = izero16
        return 0
    lax.fori_loop(0, (NGRP * _L16 + BLK) // _L16, z_body, 0)

    for r, (r_out, c_out, n_out) in enumerate(
            ((r0_out, c0_out, n0_out), (r1_out, c1_out, n1_out))):
        def grp_body(grp, carry):
            ptrv, hptr = carry
            gr0 = pl.multiple_of(row0 + grp * 8, 8)
            pltpu.sync_copy(adj_hbm.at[r, pl.ds(gr0, 8), :], abuf)
            for lr in range(8):
                rowvec = izero16 + (gr0 + lr)

                def g_body(g, ptrv):
                    vals = abuf[lr, pl.ds(g * _L16, _L16)]
                    m = vals != 0.0
                    mi = jnp.where(m, 1, 0).astype(jnp.int32)
                    pos = ptrv + plsc.cumsum(mi) - mi
                    colv = iota16 + g * _L16
                    plsc.store_scatter(rbuf, [pos], rowvec, mask=m)
                    plsc.store_scatter(cbuf, [pos], colv, mask=m)
                    return ptrv + plsc.all_reduce_population_count(m)

                ptrv = lax.fori_loop(0, NGRP, g_body, ptrv, unroll=2)

                ptr = jnp.max(ptrv)
                nfull = lax.div(ptr, BLK)

                def fl_body(k, _):
                    off = pl.multiple_of(k * BLK, BLK)
                    dst = pl.multiple_of(hptr + k * BLK, BLK)
                    pltpu.sync_copy(rbuf.at[pl.ds(off, BLK)],
                                    r_out.at[wid, 0, pl.ds(dst, BLK)])
                    pltpu.sync_copy(cbuf.at[pl.ds(off, BLK)],
                                    c_out.at[wid, 0, pl.ds(dst, BLK)])
                    return 0
                lax.fori_loop(0, nfull, fl_body, 0)

                rem = ptr - nfull * BLK
                srcb = pl.multiple_of(nfull * BLK, BLK)
                for t in range(BLK // _L16):
                    rv = rbuf[pl.ds(srcb + t * _L16, _L16)]
                    cvv = cbuf[pl.ds(srcb + t * _L16, _L16)]
                    rbuf[pl.ds(t * _L16, _L16)] = rv
                    cbuf[pl.ds(t * _L16, _L16)] = cvv
                hptr = hptr + nfull * BLK
                ptrv = izero16 + rem
            return (ptrv, hptr)

        ptrv, hptr = lax.fori_loop(0, ngrp, grp_body,
                                   (izero16, jnp.int32(0)))
        ptr = jnp.max(ptrv)
        dst = pl.multiple_of(hptr, BLK)
        pltpu.sync_copy(rbuf.at[pl.ds(0, BLK)], r_out.at[wid, 0, pl.ds(dst, BLK)])
        pltpu.sync_copy(cbuf.at[pl.ds(0, BLK)], c_out.at[wid, 0, pl.ds(dst, BLK)])
        cv[0, :] = jnp.where(iota16 == 0, hptr + ptr, 0)
        pltpu.sync_copy(cv, n_out.at[wid])


def _extract(adj_list):
    mesh = plsc.VectorSubcoreMesh(core_axis_name="c", subcore_axis_name="s")
    slab = jax.ShapeDtypeStruct((NW, 1, CAPW), jnp.int32)
    cnts = jax.ShapeDtypeStruct((NW, 1, _L16), jnp.int32)
    return pl.kernel(
        _ext_body,
        out_type=(slab, slab, cnts, slab, slab, cnts),
        mesh=mesh,
        scratch_types=[
            pltpu.VMEM((8, N), jnp.float32),
            pltpu.VMEM((NGRP * _L16 + BLK,), jnp.int32),
            pltpu.VMEM((NGRP * _L16 + BLK,), jnp.int32),
            pltpu.VMEM((1, _L16), jnp.int32),
        ],
        compiler_params=pltpu.CompilerParams(needs_layout_passes=False),
    )(adj_list)


def _deg_body(row0_hbm, n0_hbm, row1_hbm, n1_hbm, deg0_out, deg1_out,
              deg_v, ridx_v, cnt_v):
    cid = lax.axis_index("c")
    sid = lax.axis_index("s")
    wid = cid * NS + sid

    iota16 = lax.iota(jnp.int32, _L16)
    zeros16 = jnp.zeros((_L16,), jnp.float32)
    izero16 = jnp.zeros((_L16,), jnp.int32)
    for r, (row_hbm, n_hbm, deg_out) in enumerate(
            ((row0_hbm, n0_hbm, deg0_out), (row1_hbm, n1_hbm, deg1_out))):
        def zero_body(i, _):
            deg_v[0, pl.ds(i * _L16, _L16)] = zeros16
            return 0
        lax.fori_loop(0, N // _L16, zero_body, 0)

        pltpu.sync_copy(n_hbm.at[wid], cnt_v)
        cnt_w = jnp.max(cnt_v[0, :])
        nblk = lax.div(cnt_w + BLK - 1, BLK)

        def blk_body(b, _):
            off = pl.multiple_of(b * BLK, BLK)
            pltpu.sync_copy(row_hbm.at[wid, 0, pl.ds(off, BLK)], ridx_v)
            for g in range(BLK // _L16):
                r16 = ridx_v[pl.ds(g * _L16, _L16)]
                idx16 = iota16 + (b * BLK + g * _L16)
                m16 = jnp.where(idx16 < cnt_w, 1.0, 0.0)
                plsc.addupdate_scatter(deg_v, [izero16, r16], m16)
            return 0
        lax.fori_loop(0, nblk, blk_body, 0)
        pltpu.sync_copy(deg_v, deg_out.at[wid])


def _deg_kernel(row0, n0, row1, n1):
    mesh = plsc.VectorSubcoreMesh(core_axis_name="c", subcore_axis_name="s")
    return pl.kernel(
        _deg_body,
        out_type=(
            jax.ShapeDtypeStruct((NW, 1, N), jnp.float32),
            jax.ShapeDtypeStruct((NW, 1, N), jnp.float32),
        ),
        mesh=mesh,
        scratch_types=[
            pltpu.VMEM((1, N), jnp.float32),
            pltpu.VMEM((BLK,), jnp.int32),
            pltpu.VMEM((1, _L16), jnp.int32),
        ],
        compiler_params=pltpu.CompilerParams(needs_layout_passes=False),
    )(row0, n0, row1, n1)


def _edge_body(h1_hbm, ds_hbm, row_hbm, col_hbm, n_hbm,
               zeros_hbm, fout_hbm, s_hbm,
               accum, ds_v, ridxb, cidxb, cnt_v,
               hr0, hr1, hc0, hc1, rc0, rc1, cc0, cc1, s_v,
               gsr0, gsr1, gsc0, gsc1, ss0, ss1):
    cid = lax.axis_index("c")
    sid = lax.axis_index("s")
    wid = cid * NS + sid

    hr = (hr0, hr1)
    hc = (hc0, hc1)
    rcur = (rc0, rc1)
    ccur = (cc0, cc1)
    gsr = (gsr0, gsr1)
    gsc = (gsc0, gsc1)
    ssem = (ss0, ss1)

    @pl.when(sid == 0)
    def _():
        pltpu.sync_copy(zeros_hbm, accum)

    pltpu.sync_copy(ds_hbm, ds_v)
    pltpu.sync_copy(n_hbm.at[wid], cnt_v)
    cnt_w = jnp.max(cnt_v[0, :])
    nsub = lax.div(cnt_w + SUB - 1, SUB)
    nsr = lax.div(nsub + NSB - 1, NSB)
    plsc.subcore_barrier()

    iota16 = lax.iota(jnp.int32, _L16)
    z16 = jnp.zeros((_L16,), jnp.float32)
    e_base = [iota16 + g * _L16 for g in range(SUB // _L16)]

    def prep(par, jloc):
        off = jnp.minimum(jloc, NSB - 1) * SUB
        for g2 in range(SUB // _L16):
            sl = pl.ds(off + g2 * _L16, _L16)
            dsl = pl.ds(g2 * _L16, _L16)
            rcur[par][dsl] = jnp.minimum(jnp.maximum(ridxb[sl], 0), N - 1)
            ccur[par][dsl] = jnp.minimum(jnp.maximum(cidxb[sl], 0), N - 1)

    def gath(par):
        pltpu.async_copy(h1_hbm.at[rcur[par]], hr[par], gsr[par])
        pltpu.async_copy(h1_hbm.at[ccur[par]], hc[par], gsc[par])

    def gath_wait(par):
        pltpu.make_async_copy(h1_hbm.at[rcur[par]], hr[par], gsr[par]).wait()
        pltpu.make_async_copy(h1_hbm.at[ccur[par]], hc[par], gsc[par]).wait()

    def scat(par):
        pltpu.async_copy(hr[par], accum.at[ccur[par]], ssem[par], add=True)

    def scat_wait(par):
        pltpu.make_async_copy(hr[par], accum.at[ccur[par]], ssem[par]).wait()

    def compute(par, base_e, carry):
        s_num, s_den = carry
        hrv = hr[par]
        hcv = hc[par]

        def f_body(f, accs):
            fv = jnp.full((_L16,), f, dtype=jnp.int32)
            out = []
            for (d, qa, qb), e16 in zip(accs, e_base):
                a = plsc.load_gather(hrv, [e16, fv])
                b = plsc.load_gather(hcv, [e16, fv])
                out.append((d + a * b, qa + a * a, qb + b * b))
            return tuple(out)

        dots = lax.fori_loop(0, H, f_body,
                             tuple((z16, z16, z16) for _ in range(SUB // _L16)),
                             unroll=4)

        norms = []
        for g in range(SUB // _L16):
            sl = pl.ds(g * _L16, _L16)
            r16 = rcur[par][sl]
            c16 = ccur[par][sl]
            idx16 = iota16 + (base_e + g * _L16)
            m16 = jnp.where(idx16 < cnt_w, 1.0, 0.0)
            dot16, qr, qc = dots[g]
            sr = plsc.load_gather(ds_v, [r16])
            sc = plsc.load_gather(ds_v, [c16])
            d_raw = jnp.maximum(qr + qc - 2.0 * dot16, 0.0)
            d_scl = jnp.maximum(sr * sr * qr + sc * sc * qc
                                - 2.0 * (sr * sc) * dot16, 0.0)
            g_raw = _tanh_pos_sc(_rsqrt_sc(d_raw + 1e-4))
            g_scl = _tanh_pos_sc(_rsqrt_sc(d_scl + 1e-4))
            srsc = sr * sc
            norms.append(g_scl * srsc * srsc * m16)
            s_num = s_num + g_raw * m16
            s_den = s_den + m16

        def s_body(f, _):
            fv = jnp.full((_L16,), f, dtype=jnp.int32)
            for g in range(SUB // _L16):
                val = plsc.load_gather(hrv, [e_base[g], fv]) * norms[g]
                plsc.store_scatter(hrv, [e_base[g], fv], val)
            return 0
        lax.fori_loop(0, H, s_body, 0, unroll=4)
        return (s_num, s_den)

    def round_body(sr_i, carry):
        s_num, s_den = carry
        soff = pl.multiple_of(sr_i * (NSB * SUB), NSB * SUB)
        pltpu.sync_copy(row_hbm.at[wid, 0, pl.ds(soff, NSB * SUB)], ridxb)
        pltpu.sync_copy(col_hbm.at[wid, 0, pl.ds(soff, NSB * SUB)], cidxb)
        nsb_r = jnp.minimum(NSB, nsub - sr_i * NSB)
        npairs = lax.div(nsb_r + 1, 2)
        base0 = sr_i * (NSB * SUB)

        prep(0, 0)
        gath(0)

        def pair_body(t, carry):
            s_num, s_den = carry
            j0 = 2 * t
            @pl.when(t > 0)
            def _():
                scat_wait(1)
            prep(1, j0 + 1)
            gath(1)
            gath_wait(0)
            s_num, s_den = compute(0, base0 + j0 * SUB, (s_num, s_den))
            scat(0)
            scat_wait(0)
            prep(0, j0 + 2)
            gath(0)
            gath_wait(1)
            s_num, s_den = compute(1, base0 + (j0 + 1) * SUB, (s_num, s_den))
            scat(1)
            return (s_num, s_den)

        s_num, s_den = lax.fori_loop(0, npairs, pair_body, (s_num, s_den))
        scat_wait(1)
        gath_wait(0)
        return (s_num, s_den)

    s_num, s_den = lax.fori_loop(0, nsr, round_body, (z16, z16))

    svec = jnp.where(iota16 == 0, jnp.sum(s_num),
                     jnp.where(iota16 == 1, jnp.sum(s_den), 0.0))
    s_v[0, :] = svec
    pltpu.sync_copy(s_v, s_hbm.at[wid])

    plsc.subcore_barrier()

    @pl.when(sid < NS - 1)
    def _():
        pltpu.sync_copy(accum.at[pl.ds(sid * RPT, RPT)],
                        fout_hbm.at[cid, pl.ds(sid * RPT, RPT)])

    @pl.when(sid == NS - 1)
    def _():
        pltpu.sync_copy(accum.at[pl.ds((NS - 1) * RPT, N - (NS - 1) * RPT)],
                        fout_hbm.at[cid, pl.ds((NS - 1) * RPT, N - (NS - 1) * RPT)])


def _edge_kernel(h1, ds_r, row_r, col_r, n_r, zeros):
    mesh = plsc.VectorSubcoreMesh(core_axis_name="c", subcore_axis_name="s")
    return pl.kernel(
        _edge_body,
        out_type=(
            jax.ShapeDtypeStruct((NC, N, H), jnp.float32),
            jax.ShapeDtypeStruct((NW, 1, _L16), jnp.float32),
        ),
        mesh=mesh,
        scratch_types=[
            pltpu.VMEM_SHARED((N, H), jnp.float32),
            pltpu.VMEM((N,), jnp.float32),
            pltpu.VMEM((NSB * SUB,), jnp.int32),
            pltpu.VMEM((NSB * SUB,), jnp.int32),
            pltpu.VMEM((1, _L16), jnp.int32),
            pltpu.VMEM((SUB, H), jnp.float32),
            pltpu.VMEM((SUB, H), jnp.float32),
            pltpu.VMEM((SUB, H), jnp.float32),
            pltpu.VMEM((SUB, H), jnp.float32),
            pltpu.VMEM((SUB,), jnp.int32),
            pltpu.VMEM((SUB,), jnp.int32),
            pltpu.VMEM((SUB,), jnp.int32),
            pltpu.VMEM((SUB,), jnp.int32),
            pltpu.VMEM((1, _L16), jnp.float32),
            pltpu.SemaphoreType.DMA,
            pltpu.SemaphoreType.DMA,
            pltpu.SemaphoreType.DMA,
            pltpu.SemaphoreType.DMA,
            pltpu.SemaphoreType.DMA,
            pltpu.SemaphoreType.DMA,
        ],
        compiler_params=pltpu.CompilerParams(needs_layout_passes=False),
    )(h1, ds_r, row_r, col_r, n_r, zeros)


MB = 1000
MG = N // MB


def _mm1_body(h_ref, w_ref, b_ref, h1_ref):
    h1 = jnp.dot(h_ref[...], w_ref[...], preferred_element_type=jnp.float32)
    h1_ref[...] = jnp.maximum(h1 + b_ref[...], 0.0)


def _mm1(h, w1t, b1):
    return pl.pallas_call(
        _mm1_body,
        grid=(MG,),
        in_specs=[
            pl.BlockSpec((MB, F_IN), lambda i: (i, 0)),
            pl.BlockSpec((F_IN, H), lambda i: (0, 0)),
            pl.BlockSpec((1, H), lambda i: (0, 0)),
        ],
        out_specs=pl.BlockSpec((MB, H), lambda i: (i, 0)),
        out_shape=jax.ShapeDtypeStruct((N, H), jnp.float32),
    )(h, w1t, b1)


def _prep_body(p0_ref, p1_ref, ds_ref):
    d0 = jnp.sum(p0_ref[...], axis=(0, 1))
    d1 = jnp.sum(p1_ref[...], axis=(0, 1))
    deg = jnp.maximum(jnp.stack([d0, d1]), 1.0)
    ds_ref[...] = lax.sqrt(lax.rsqrt(deg))


def _prep(deg0, deg1):
    return pl.pallas_call(
        _prep_body,
        out_shape=jax.ShapeDtypeStruct((R, N), jnp.float32),
    )(deg0, deg1)


def _combine_body(raw_ref, f0_ref, f1_ref, s0_ref, s1_ref, h1_ref):
    s0 = s0_ref[...]
    s1 = s1_ref[...]
    sr0 = jnp.sum(s0[:, 0, 0]) / jnp.sum(s0[:, 0, 1])
    sr1 = jnp.sum(s1[:, 0, 0]) / jnp.sum(s1[:, 0, 1])
    l1tr = jnp.abs(sr0) + jnp.abs(sr1)
    fi = l1tr + 2.0 * LAMDA2 / LAMDA1
    cl = 2.0 * LAMDA2 / LAMDA1
    u0 = jnp.float32(0.5)
    u1 = jnp.float32(0.5)
    for it in range(ITERS):
        t_t = np.float32(np.sqrt(2.0 * np.log(R) / (it + 1.0))) / fi
        u0t = u0 * jnp.exp(-t_t * (cl * u0 + sr0))
        u1t = u1 * jnp.exp(-t_t * (cl * u1 + sr1))
        ssum = u0t + u1t
        u0 = u0t / ssum
        u1 = u1t / ssum
    f_sum = u0 * (f0_ref[0] + f0_ref[1]) + u1 * (f1_ref[0] + f1_ref[1])
    h1_ref[...] = ALPHA * raw_ref[...] + (1.0 - ALPHA) * f_sum


def _combine(raw, f0, f1, s0, s1):
    return pl.pallas_call(
        _combine_body,
        grid=(MG,),
        in_specs=[
            pl.BlockSpec((MB, H), lambda i: (i, 0)),
            pl.BlockSpec((NC, MB, H), lambda i: (0, i, 0)),
            pl.BlockSpec((NC, MB, H), lambda i: (0, i, 0)),
            pl.BlockSpec((NW, 1, _L16), lambda i: (0, 0, 0)),
            pl.BlockSpec((NW, 1, _L16), lambda i: (0, 0, 0)),
        ],
        out_specs=pl.BlockSpec((MB, H), lambda i: (i, 0)),
        out_shape=jax.ShapeDtypeStruct((N, H), jnp.float32),
    )(raw, f0, f1, s0, s1)


def _final_body(h1_ref, w_ref, b_ref, lp_ref, lg_ref):
    logits = jnp.dot(h1_ref[...], w_ref[...], preferred_element_type=jnp.float32)
    logits = logits + b_ref[...]
    m = jnp.max(logits, axis=1, keepdims=True)
    lse = m + jnp.log(jnp.sum(jnp.exp(logits - m), axis=1, keepdims=True))
    lg_ref[...] = logits
    lp_ref[...] = logits - lse


def _final(h1, w2t, b2):
    return pl.pallas_call(
        _final_body,
        grid=(MG,),
        in_specs=[
            pl.BlockSpec((MB, H), lambda i: (i, 0)),
            pl.BlockSpec((H, C), lambda i: (0, 0)),
            pl.BlockSpec((1, C), lambda i: (0, 0)),
        ],
        out_specs=(
            pl.BlockSpec((MB, C), lambda i: (i, 0)),
            pl.BlockSpec((MB, C), lambda i: (i, 0)),
        ),
        out_shape=(
            jax.ShapeDtypeStruct((N, C), jnp.float32),
            jax.ShapeDtypeStruct((N, C), jnp.float32),
        ),
    )(h1, w2t, b2)


def kernel(h, adj_list, labels, W1, b1, W2, b2):
    r0, c0, n0, r1, c1, n1 = _extract(adj_list)
    rows, cols, cnts = (r0, r1), (c0, c1), (n0, n1)

    h1 = _mm1(h, W1.T, b1.reshape(1, H))
    raw = h1
    deg0, deg1 = _deg_kernel(rows[0], cnts[0], rows[1], cnts[1])
    ds = _prep(deg0, deg1)
    zeros = jnp.zeros((N, H), jnp.float32)

    for _ in range(LAYER_NUM):
        f_list, s_list = [], []
        for r in range(R):
            fpart, spart = _edge_kernel(h1, ds[r],
                                        rows[r], cols[r], cnts[r], zeros)
            f_list.append(fpart)
            s_list.append(spart)
        h1 = _combine(raw, f_list[0], f_list[1], s_list[0], s_list[1])

    return _final(h1, W2.T, b2.reshape(1, C))


# speedup vs baseline: 4.4458x; 1.0008x over previous
import functools

import numpy as np
import jax
import jax.numpy as jnp
from jax import lax
from jax.experimental import pallas as pl
from jax.experimental.pallas import tpu as pltpu
from jax.experimental.pallas import tpu_sc as plsc

N = 10000
R = 2
E = 160000
F_IN = 256
H = 128
C = 16
ALPHA = 0.1
LAMDA2 = 0.01
LAMDA1 = 1.0 / ALPHA - 1.0
LAYER_NUM = 2
ITERS = 15

NC = 2
NS = 16
NW = NC * NS
BLK = 128
SUB = 64
NSB = 32
RPT = 640
RPW = 320
CAPW = 161792
NGRP = 625

_L16 = 16


def _rsqrt_sc(x):
    i = plsc.bitcast(x, jnp.int32)
    y = plsc.bitcast(jnp.int32(0x5F3759DF) - (i >> 1), jnp.float32)
    for _ in range(3):
        y = y * (1.5 - 0.5 * x * y * y)
    return y


def _tanh_pos_sc(y):
    t = jnp.exp(-2.0 * y)
    return (1.0 - t) / (1.0 + t)


def _ext_body(adj_hbm, r0_out, c0_out, n0_out, r1_out, c1_out, n1_out,
              abuf, rbuf, cbuf, cv):
    cid = lax.axis_index("c")
    sid = lax.axis_index("s")
    wid = cid * NS + sid
    row0 = wid * RPW
    nrows = jnp.minimum(RPW, N - row0)
    ngrp = nrows // 8

    iota16 = lax.iota(jnp.int32, _L16)
    izero16 = jnp.zeros((_L16,), jnp.int32)

    def z_body(i, _):
        rbuf[pl.ds(i * _L16, _L16)] = izero16
        cbuf[pl.ds(i * _L16, _L16)] = izero16
        return 0
    lax.fori_loop(0, (NGRP * _L16 + BLK) // _L16, z_body, 0)

    for r, (r_out, c_out, n_out) in enumerate(
            ((r0_out, c0_out, n0_out), (r1_out, c1_out, n1_out))):
        def grp_body(grp, carry):
            ptrv, hptr = carry
            gr0 = pl.multiple_of(row0 + grp * 8, 8)
            pltpu.sync_copy(adj_hbm.at[r, pl.ds(gr0, 8), :], abuf)
            for lr in range(8):
                rowvec = izero16 + (gr0 + lr)

                def g_body(g, ptrv):
                    vals = abuf[lr, pl.ds(g * _L16, _L16)]
                    m = vals != 0.0
                    mi = jnp.where(m, 1, 0).astype(jnp.int32)
                    pos = ptrv + plsc.cumsum(mi) - mi
                    colv = iota16 + g * _L16
                    plsc.store_scatter(rbuf, [pos], rowvec, mask=m)
                    plsc.store_scatter(cbuf, [pos], colv, mask=m)
                    return ptrv + plsc.all_reduce_population_count(m)

                ptrv = lax.fori_loop(0, NGRP, g_body, ptrv, unroll=2)

                ptr = jnp.max(ptrv)
                nfull = lax.div(ptr, BLK)

                def fl_body(k, _):
                    off = pl.multiple_of(k * BLK, BLK)
                    dst = pl.multiple_of(hptr + k * BLK, BLK)
                    pltpu.sync_copy(rbuf.at[pl.ds(off, BLK)],
                                    r_out.at[wid, 0, pl.ds(dst, BLK)])
                    pltpu.sync_copy(cbuf.at[pl.ds(off, BLK)],
                                    c_out.at[wid, 0, pl.ds(dst, BLK)])
                    return 0
                lax.fori_loop(0, nfull, fl_body, 0)

                rem = ptr - nfull * BLK
                srcb = pl.multiple_of(nfull * BLK, BLK)
                for t in range(BLK // _L16):
                    rv = rbuf[pl.ds(srcb + t * _L16, _L16)]
                    cvv = cbuf[pl.ds(srcb + t * _L16, _L16)]
                    rbuf[pl.ds(t * _L16, _L16)] = rv
                    cbuf[pl.ds(t * _L16, _L16)] = cvv
                hptr = hptr + nfull * BLK
                ptrv = izero16 + rem
            return (ptrv, hptr)

        ptrv, hptr = lax.fori_loop(0, ngrp, grp_body,
                                   (izero16, jnp.int32(0)))
        ptr = jnp.max(ptrv)
        dst = pl.multiple_of(hptr, BLK)
        pltpu.sync_copy(rbuf.at[pl.ds(0, BLK)], r_out.at[wid, 0, pl.ds(dst, BLK)])
        pltpu.sync_copy(cbuf.at[pl.ds(0, BLK)], c_out.at[wid, 0, pl.ds(dst, BLK)])
        cv[0, :] = jnp.where(iota16 == 0, hptr + ptr, 0)
        pltpu.sync_copy(cv, n_out.at[wid])


def _extract(adj_list):
    mesh = plsc.VectorSubcoreMesh(core_axis_name="c", subcore_axis_name="s")
    slab = jax.ShapeDtypeStruct((NW, 1, CAPW), jnp.int32)
    cnts = jax.ShapeDtypeStruct((NW, 1, _L16), jnp.int32)
    return pl.kernel(
        _ext_body,
        out_type=(slab, slab, cnts, slab, slab, cnts),
        mesh=mesh,
        scratch_types=[
            pltpu.VMEM((8, N), jnp.float32),
            pltpu.VMEM((NGRP * _L16 + BLK,), jnp.int32),
            pltpu.VMEM((NGRP * _L16 + BLK,), jnp.int32),
            pltpu.VMEM((1, _L16), jnp.int32),
        ],
        compiler_params=pltpu.CompilerParams(needs_layout_passes=False),
    )(adj_list)


def _deg_body(row0_hbm, n0_hbm, row1_hbm, n1_hbm, deg0_out, deg1_out,
              deg_v, ridx_v, cnt_v):
    cid = lax.axis_index("c")
    sid = lax.axis_index("s")
    wid = cid * NS + sid

    iota16 = lax.iota(jnp.int32, _L16)
    zeros16 = jnp.zeros((_L16,), jnp.float32)
    izero16 = jnp.zeros((_L16,), jnp.int32)
    for r, (row_hbm, n_hbm, deg_out) in enumerate(
            ((row0_hbm, n0_hbm, deg0_out), (row1_hbm, n1_hbm, deg1_out))):
        def zero_body(i, _):
            deg_v[0, pl.ds(i * _L16, _L16)] = zeros16
            return 0
        lax.fori_loop(0, N // _L16, zero_body, 0)

        pltpu.sync_copy(n_hbm.at[wid], cnt_v)
        cnt_w = jnp.max(cnt_v[0, :])
        nblk = lax.div(cnt_w + BLK - 1, BLK)

        def blk_body(b, _):
            off = pl.multiple_of(b * BLK, BLK)
            pltpu.sync_copy(row_hbm.at[wid, 0, pl.ds(off, BLK)], ridx_v)
            for g in range(BLK // _L16):
                r16 = ridx_v[pl.ds(g * _L16, _L16)]
                idx16 = iota16 + (b * BLK + g * _L16)
                m16 = jnp.where(idx16 < cnt_w, 1.0, 0.0)
                plsc.addupdate_scatter(deg_v, [izero16, r16], m16)
            return 0
        lax.fori_loop(0, nblk, blk_body, 0)
        pltpu.sync_copy(deg_v, deg_out.at[wid])


def _deg_kernel(row0, n0, row1, n1):
    mesh = plsc.VectorSubcoreMesh(core_axis_name="c", subcore_axis_name="s")
    return pl.kernel(
        _deg_body,
        out_type=(
            jax.ShapeDtypeStruct((NW, 1, N), jnp.float32),
            jax.ShapeDtypeStruct((NW, 1, N), jnp.float32),
        ),
        mesh=mesh,
        scratch_types=[
            pltpu.VMEM((1, N), jnp.float32),
            pltpu.VMEM((BLK,), jnp.int32),
            pltpu.VMEM((1, _L16), jnp.int32),
        ],
        compiler_params=pltpu.CompilerParams(needs_layout_passes=False),
    )(row0, n0, row1, n1)


def _edge_body(h1_hbm, ds_hbm, row_hbm, col_hbm, n_hbm,
               zeros_hbm, fout_hbm, s_hbm,
               accum, ds_v, ridxb, cidxb, cnt_v,
               hr0, hr1, hc0, hc1, rc0, rc1, cc0, cc1, s_v,
               gsr0, gsr1, gsc0, gsc1, gsr0b, gsr1b, gsc0b, gsc1b, ss0, ss1):
    cid = lax.axis_index("c")
    sid = lax.axis_index("s")
    wid = cid * NS + sid

    hr = (hr0, hr1)
    hc = (hc0, hc1)
    rcur = (rc0, rc1)
    ccur = (cc0, cc1)
    gsr = (gsr0, gsr1)
    gsc = (gsc0, gsc1)
    gsrb = (gsr0b, gsr1b)
    gscb = (gsc0b, gsc1b)
    ssem = (ss0, ss1)
    HS = SUB // 2

    @pl.when(sid == 0)
    def _():
        pltpu.sync_copy(zeros_hbm, accum)

    pltpu.sync_copy(ds_hbm, ds_v)
    pltpu.sync_copy(n_hbm.at[wid], cnt_v)
    cnt_w = jnp.max(cnt_v[0, :])
    nsub = lax.div(cnt_w + SUB - 1, SUB)
    nsr = lax.div(nsub + NSB - 1, NSB)
    plsc.subcore_barrier()

    iota16 = lax.iota(jnp.int32, _L16)
    z16 = jnp.zeros((_L16,), jnp.float32)
    e_base = [iota16 + g * _L16 for g in range(SUB // _L16)]

    def prep(par, jloc):
        off = jnp.minimum(jloc, NSB - 1) * SUB
        for g2 in range(SUB // _L16):
            sl = pl.ds(off + g2 * _L16, _L16)
            dsl = pl.ds(g2 * _L16, _L16)
            rcur[par][dsl] = jnp.minimum(jnp.maximum(ridxb[sl], 0), N - 1)
            ccur[par][dsl] = jnp.minimum(jnp.maximum(cidxb[sl], 0), N - 1)

    def _gparts(par):
        return (
            (h1_hbm.at[rcur[par].at[pl.ds(0, HS)]], hr[par].at[pl.ds(0, HS)], gsr[par]),
            (h1_hbm.at[rcur[par].at[pl.ds(HS, HS)]], hr[par].at[pl.ds(HS, HS)], gsrb[par]),
            (h1_hbm.at[ccur[par].at[pl.ds(0, HS)]], hc[par].at[pl.ds(0, HS)], gsc[par]),
            (h1_hbm.at[ccur[par].at[pl.ds(HS, HS)]], hc[par].at[pl.ds(HS, HS)], gscb[par]),
        )

    def gath(par):
        for s_, d_, m_ in _gparts(par):
            pltpu.async_copy(s_, d_, m_)

    def gath_wait(par):
        for s_, d_, m_ in _gparts(par):
            pltpu.make_async_copy(s_, d_, m_).wait()

    def scat(par):
        pltpu.async_copy(hr[par], accum.at[ccur[par]], ssem[par], add=True)

    def scat_wait(par):
        pltpu.make_async_copy(hr[par], accum.at[ccur[par]], ssem[par]).wait()

    def compute(par, base_e, carry):
        s_num, s_den = carry
        hrv = hr[par]
        hcv = hc[par]

        def f_body(f, accs):
            fv = jnp.full((_L16,), f, dtype=jnp.int32)
            out = []
            for (d, qa, qb), e16 in zip(accs, e_base):
                a = plsc.load_gather(hrv, [e16, fv])
                b = plsc.load_gather(hcv, [e16, fv])
                out.append((d + a * b, qa + a * a, qb + b * b))
            return tuple(out)

        dots = lax.fori_loop(0, H, f_body,
                             tuple((z16, z16, z16) for _ in range(SUB // _L16)),
                             unroll=4)

        norms = []
        for g in range(SUB // _L16):
            sl = pl.ds(g * _L16, _L16)
            r16 = rcur[par][sl]
            c16 = ccur[par][sl]
            idx16 = iota16 + (base_e + g * _L16)
            m16 = jnp.where(idx16 < cnt_w, 1.0, 0.0)
            dot16, qr, qc = dots[g]
            sr = plsc.load_gather(ds_v, [r16])
            sc = plsc.load_gather(ds_v, [c16])
            d_raw = jnp.maximum(qr + qc - 2.0 * dot16, 0.0)
            d_scl = jnp.maximum(sr * sr * qr + sc * sc * qc
                                - 2.0 * (sr * sc) * dot16, 0.0)
            g_raw = _tanh_pos_sc(_rsqrt_sc(d_raw + 1e-4))
            g_scl = _tanh_pos_sc(_rsqrt_sc(d_scl + 1e-4))
            srsc = sr * sc
            norms.append(g_scl * srsc * srsc * m16)
            s_num = s_num + g_raw * m16
            s_den = s_den + m16

        def s_body(f, _):
            fv = jnp.full((_L16,), f, dtype=jnp.int32)
            for g in range(SUB // _L16):
                val = plsc.load_gather(hrv, [e_base[g], fv]) * norms[g]
                plsc.store_scatter(hrv, [e_base[g], fv], val)
            return 0
        lax.fori_loop(0, H, s_body, 0, unroll=4)
        return (s_num, s_den)

    def round_body(sr_i, carry):
        s_num, s_den = carry
        soff = pl.multiple_of(sr_i * (NSB * SUB), NSB * SUB)
        pltpu.sync_copy(row_hbm.at[wid, 0, pl.ds(soff, NSB * SUB)], ridxb)
        pltpu.sync_copy(col_hbm.at[wid, 0, pl.ds(soff, NSB * SUB)], cidxb)
        nsb_r = jnp.minimum(NSB, nsub - sr_i * NSB)
        npairs = lax.div(nsb_r + 1, 2)
        base0 = sr_i * (NSB * SUB)

        prep(0, 0)
        gath(0)

        def pair_body(t, carry):
            s_num, s_den = carry
            j0 = 2 * t
            @pl.when(t > 0)
            def _():
                scat_wait(1)
            prep(1, j0 + 1)
            gath(1)
            gath_wait(0)
            s_num, s_den = compute(0, base0 + j0 * SUB, (s_num, s_den))
            scat(0)
            scat_wait(0)
            prep(0, j0 + 2)
            gath(0)
            gath_wait(1)
            s_num, s_den = compute(1, base0 + (j0 + 1) * SUB, (s_num, s_den))
            scat(1)
            return (s_num, s_den)

        s_num, s_den = lax.fori_loop(0, npairs, pair_body, (s_num, s_den))
        scat_wait(1)
        gath_wait(0)
        return (s_num, s_den)

    s_num, s_den = lax.fori_loop(0, nsr, round_body, (z16, z16))

    svec = jnp.where(iota16 == 0, jnp.sum(s_num),
                     jnp.where(iota16 == 1, jnp.sum(s_den), 0.0))
    s_v[0, :] = svec
    pltpu.sync_copy(s_v, s_hbm.at[wid])

    plsc.subcore_barrier()

    @pl.when(sid < NS - 1)
    def _():
        pltpu.sync_copy(accum.at[pl.ds(sid * RPT, RPT)],
                        fout_hbm.at[cid, pl.ds(sid * RPT, RPT)])

    @pl.when(sid == NS - 1)
    def _():
        pltpu.sync_copy(accum.at[pl.ds((NS - 1) * RPT, N - (NS - 1) * RPT)],
                        fout_hbm.at[cid, pl.ds((NS - 1) * RPT, N - (NS - 1) * RPT)])


def _edge_kernel(h1, ds_r, row_r, col_r, n_r, zeros):
    mesh = plsc.VectorSubcoreMesh(core_axis_name="c", subcore_axis_name="s")
    return pl.kernel(
        _edge_body,
        out_type=(
            jax.ShapeDtypeStruct((NC, N, H), jnp.float32),
            jax.ShapeDtypeStruct((NW, 1, _L16), jnp.float32),
        ),
        mesh=mesh,
        scratch_types=[
            pltpu.VMEM_SHARED((N, H), jnp.float32),
            pltpu.VMEM((N,), jnp.float32),
            pltpu.VMEM((NSB * SUB,), jnp.int32),
            pltpu.VMEM((NSB * SUB,), jnp.int32),
            pltpu.VMEM((1, _L16), jnp.int32),
            pltpu.VMEM((SUB, H), jnp.float32),
            pltpu.VMEM((SUB, H), jnp.float32),
            pltpu.VMEM((SUB, H), jnp.float32),
            pltpu.VMEM((SUB, H), jnp.float32),
            pltpu.VMEM((SUB,), jnp.int32),
            pltpu.VMEM((SUB,), jnp.int32),
            pltpu.VMEM((SUB,), jnp.int32),
            pltpu.VMEM((SUB,), jnp.int32),
            pltpu.VMEM((1, _L16), jnp.float32),
            pltpu.SemaphoreType.DMA,
            pltpu.SemaphoreType.DMA,
            pltpu.SemaphoreType.DMA,
            pltpu.SemaphoreType.DMA,
            pltpu.SemaphoreType.DMA,
            pltpu.SemaphoreType.DMA,
            pltpu.SemaphoreType.DMA,
            pltpu.SemaphoreType.DMA,
            pltpu.SemaphoreType.DMA,
            pltpu.SemaphoreType.DMA,
        ],
        compiler_params=pltpu.CompilerParams(needs_layout_passes=False),
    )(h1, ds_r, row_r, col_r, n_r, zeros)


MB = 1000
MG = N // MB


def _mm1_body(h_ref, w_ref, b_ref, h1_ref):
    h1 = jnp.dot(h_ref[...], w_ref[...], preferred_element_type=jnp.float32)
    h1_ref[...] = jnp.maximum(h1 + b_ref[...], 0.0)


def _mm1(h, w1t, b1):
    return pl.pallas_call(
        _mm1_body,
        grid=(MG,),
        in_specs=[
            pl.BlockSpec((MB, F_IN), lambda i: (i, 0)),
            pl.BlockSpec((F_IN, H), lambda i: (0, 0)),
            pl.BlockSpec((1, H), lambda i: (0, 0)),
        ],
        out_specs=pl.BlockSpec((MB, H), lambda i: (i, 0)),
        out_shape=jax.ShapeDtypeStruct((N, H), jnp.float32),
    )(h, w1t, b1)


def _prep_body(p0_ref, p1_ref, ds_ref):
    d0 = jnp.sum(p0_ref[...], axis=(0, 1))
    d1 = jnp.sum(p1_ref[...], axis=(0, 1))
    deg = jnp.maximum(jnp.stack([d0, d1]), 1.0)
    ds_ref[...] = lax.sqrt(lax.rsqrt(deg))


def _prep(deg0, deg1):
    return pl.pallas_call(
        _prep_body,
        out_shape=jax.ShapeDtypeStruct((R, N), jnp.float32),
    )(deg0, deg1)


def _combine_body(raw_ref, f0_ref, f1_ref, s0_ref, s1_ref, h1_ref):
    s0 = s0_ref[...]
    s1 = s1_ref[...]
    sr0 = jnp.sum(s0[:, 0, 0]) / jnp.sum(s0[:, 0, 1])
    sr1 = jnp.sum(s1[:, 0, 0]) / jnp.sum(s1[:, 0, 1])
    l1tr = jnp.abs(sr0) + jnp.abs(sr1)
    fi = l1tr + 2.0 * LAMDA2 / LAMDA1
    cl = 2.0 * LAMDA2 / LAMDA1
    u0 = jnp.float32(0.5)
    u1 = jnp.float32(0.5)
    for it in range(ITERS):
        t_t = np.float32(np.sqrt(2.0 * np.log(R) / (it + 1.0))) / fi
        u0t = u0 * jnp.exp(-t_t * (cl * u0 + sr0))
        u1t = u1 * jnp.exp(-t_t * (cl * u1 + sr1))
        ssum = u0t + u1t
        u0 = u0t / ssum
        u1 = u1t / ssum
    f_sum = u0 * (f0_ref[0] + f0_ref[1]) + u1 * (f1_ref[0] + f1_ref[1])
    h1_ref[...] = ALPHA * raw_ref[...] + (1.0 - ALPHA) * f_sum


def _combine(raw, f0, f1, s0, s1):
    return pl.pallas_call(
        _combine_body,
        grid=(MG,),
        in_specs=[
            pl.BlockSpec((MB, H), lambda i: (i, 0)),
            pl.BlockSpec((NC, MB, H), lambda i: (0, i, 0)),
            pl.BlockSpec((NC, MB, H), lambda i: (0, i, 0)),
            pl.BlockSpec((NW, 1, _L16), lambda i: (0, 0, 0)),
            pl.BlockSpec((NW, 1, _L16), lambda i: (0, 0, 0)),
        ],
        out_specs=pl.BlockSpec((MB, H), lambda i: (i, 0)),
        out_shape=jax.ShapeDtypeStruct((N, H), jnp.float32),
    )(raw, f0, f1, s0, s1)


def _final_body(h1_ref, w_ref, b_ref, lp_ref, lg_ref):
    logits = jnp.dot(h1_ref[...], w_ref[...], preferred_element_type=jnp.float32)
    logits = logits + b_ref[...]
    m = jnp.max(logits, axis=1, keepdims=True)
    lse = m + jnp.log(jnp.sum(jnp.exp(logits - m), axis=1, keepdims=True))
    lg_ref[...] = logits
    lp_ref[...] = logits - lse


def _final(h1, w2t, b2):
    return pl.pallas_call(
        _final_body,
        grid=(MG,),
        in_specs=[
            pl.BlockSpec((MB, H), lambda i: (i, 0)),
            pl.BlockSpec((H, C), lambda i: (0, 0)),
            pl.BlockSpec((1, C), lambda i: (0, 0)),
        ],
        out_specs=(
            pl.BlockSpec((MB, C), lambda i: (i, 0)),
            pl.BlockSpec((MB, C), lambda i: (i, 0)),
        ),
        out_shape=(
            jax.ShapeDtypeStruct((N, C), jnp.float32),
            jax.ShapeDtypeStruct((N, C), jnp.float32),
        ),
    )(h1, w2t, b2)


def kernel(h, adj_list, labels, W1, b1, W2, b2):
    r0, c0, n0, r1, c1, n1 = _extract(adj_list)
    rows, cols, cnts = (r0, r1), (c0, c1), (n0, n1)

    h1 = _mm1(h, W1.T, b1.reshape(1, H))
    raw = h1
    deg0, deg1 = _deg_kernel(rows[0], cnts[0], rows[1], cnts[1])
    ds = _prep(deg0, deg1)
    zeros = jnp.zeros((N, H), jnp.float32)

    for _ in range(LAYER_NUM):
        f_list, s_list = [], []
        for r in range(R):
            fpart, spart = _edge_kernel(h1, ds[r],
                                        rows[r], cols[r], cnts[r], zeros)
            f_list.append(fpart)
            s_list.append(spart)
        h1 = _combine(raw, f_list[0], f_list[1], s_list[0], s_list[1])

    return _final(h1, W2.T, b2.reshape(1, C))


# speedup vs baseline: 7.8539x; 1.7666x over previous
import functools

import numpy as np
import jax
import jax.numpy as jnp
from jax import lax
from jax.experimental import pallas as pl
from jax.experimental.pallas import tpu as pltpu
from jax.experimental.pallas import tpu_sc as plsc

N = 10000
R = 2
E = 160000
F_IN = 256
H = 128
C = 16
ALPHA = 0.1
LAMDA2 = 0.01
LAMDA1 = 1.0 / ALPHA - 1.0
LAYER_NUM = 2
ITERS = 15

NC = 2
NS = 16
NW = NC * NS
BLK = 128
SUB = 64
NSB = 32
RPT = 640
RPW = 320
CAPW = 161792
NGRP = 625

_L16 = 16


def _rsqrt_sc(x):
    i = plsc.bitcast(x, jnp.int32)
    y = plsc.bitcast(jnp.int32(0x5F3759DF) - (i >> 1), jnp.float32)
    for _ in range(3):
        y = y * (1.5 - 0.5 * x * y * y)
    return y


def _tanh_pos_sc(y):
    t = jnp.exp(-2.0 * y)
    return (1.0 - t) / (1.0 + t)


def _ext_body(adj_hbm, r0_out, c0_out, n0_out, r1_out, c1_out, n1_out,
              abuf, rbuf, cbuf, cv):
    cid = lax.axis_index("c")
    sid = lax.axis_index("s")
    wid = cid * NS + sid
    row0 = wid * RPW
    nrows = jnp.minimum(RPW, N - row0)
    ngrp = nrows // 8

    iota16 = lax.iota(jnp.int32, _L16)
    izero16 = jnp.zeros((_L16,), jnp.int32)

    def z_body(i, _):
        rbuf[pl.ds(i * _L16, _L16)] = izero16
        cbuf[pl.ds(i * _L16, _L16)] = izero16
        return 0
    lax.fori_loop(0, (NGRP * _L16 + BLK) // _L16, z_body, 0)

    for r, (r_out, c_out, n_out) in enumerate(
            ((r0_out, c0_out, n0_out), (r1_out, c1_out, n1_out))):
        def grp_body(grp, carry):
            ptrv, hptr = carry
            gr0 = pl.multiple_of(row0 + grp * 8, 8)
            pltpu.sync_copy(adj_hbm.at[r, pl.ds(gr0, 8), :], abuf)
            for lr in range(8):
                rowvec = izero16 + (gr0 + lr)

                def g_body(g, ptrv):
                    vals = abuf[lr, pl.ds(g * _L16, _L16)]
                    m = vals != 0.0
                    mi = jnp.where(m, 1, 0).astype(jnp.int32)
                    pos = ptrv + plsc.cumsum(mi) - mi
                    colv = iota16 + g * _L16
                    plsc.store_scatter(rbuf, [pos], rowvec, mask=m)
                    plsc.store_scatter(cbuf, [pos], colv, mask=m)
                    return ptrv + plsc.all_reduce_population_count(m)

                ptrv = lax.fori_loop(0, NGRP, g_body, ptrv, unroll=2)

                ptr = jnp.max(ptrv)
                nfull = lax.div(ptr, BLK)

                def fl_body(k, _):
                    off = pl.multiple_of(k * BLK, BLK)
                    dst = pl.multiple_of(hptr + k * BLK, BLK)
                    pltpu.sync_copy(rbuf.at[pl.ds(off, BLK)],
                                    r_out.at[wid, 0, pl.ds(dst, BLK)])
                    pltpu.sync_copy(cbuf.at[pl.ds(off, BLK)],
                                    c_out.at[wid, 0, pl.ds(dst, BLK)])
                    return 0
                lax.fori_loop(0, nfull, fl_body, 0)

                rem = ptr - nfull * BLK
                srcb = pl.multiple_of(nfull * BLK, BLK)
                for t in range(BLK // _L16):
                    rv = rbuf[pl.ds(srcb + t * _L16, _L16)]
                    cvv = cbuf[pl.ds(srcb + t * _L16, _L16)]
                    rbuf[pl.ds(t * _L16, _L16)] = rv
                    cbuf[pl.ds(t * _L16, _L16)] = cvv
                hptr = hptr + nfull * BLK
                ptrv = izero16 + rem
            return (ptrv, hptr)

        ptrv, hptr = lax.fori_loop(0, ngrp, grp_body,
                                   (izero16, jnp.int32(0)))
        ptr = jnp.max(ptrv)
        dst = pl.multiple_of(hptr, BLK)
        pltpu.sync_copy(rbuf.at[pl.ds(0, BLK)], r_out.at[wid, 0, pl.ds(dst, BLK)])
        pltpu.sync_copy(cbuf.at[pl.ds(0, BLK)], c_out.at[wid, 0, pl.ds(dst, BLK)])
        cv[0, :] = jnp.where(iota16 == 0, hptr + ptr, 0)
        pltpu.sync_copy(cv, n_out.at[wid])


def _extract(adj_list):
    mesh = plsc.VectorSubcoreMesh(core_axis_name="c", subcore_axis_name="s")
    slab = jax.ShapeDtypeStruct((NW, 1, CAPW), jnp.int32)
    cnts = jax.ShapeDtypeStruct((NW, 1, _L16), jnp.int32)
    return pl.kernel(
        _ext_body,
        out_type=(slab, slab, cnts, slab, slab, cnts),
        mesh=mesh,
        scratch_types=[
            pltpu.VMEM((8, N), jnp.float32),
            pltpu.VMEM((NGRP * _L16 + BLK,), jnp.int32),
            pltpu.VMEM((NGRP * _L16 + BLK,), jnp.int32),
            pltpu.VMEM((1, _L16), jnp.int32),
        ],
        compiler_params=pltpu.CompilerParams(needs_layout_passes=False),
    )(adj_list)


def _deg_body(row0_hbm, n0_hbm, row1_hbm, n1_hbm, deg0_out, deg1_out,
              deg_v, ridx_v, cnt_v):
    cid = lax.axis_index("c")
    sid = lax.axis_index("s")
    wid = cid * NS + sid

    iota16 = lax.iota(jnp.int32, _L16)
    zeros16 = jnp.zeros((_L16,), jnp.float32)
    izero16 = jnp.zeros((_L16,), jnp.int32)
    for r, (row_hbm, n_hbm, deg_out) in enumerate(
            ((row0_hbm, n0_hbm, deg0_out), (row1_hbm, n1_hbm, deg1_out))):
        def zero_body(i, _):
            deg_v[0, pl.ds(i * _L16, _L16)] = zeros16
            return 0
        lax.fori_loop(0, N // _L16, zero_body, 0)

        pltpu.sync_copy(n_hbm.at[wid], cnt_v)
        cnt_w = jnp.max(cnt_v[0, :])
        nblk = lax.div(cnt_w + BLK - 1, BLK)

        def blk_body(b, _):
            off = pl.multiple_of(b * BLK, BLK)
            pltpu.sync_copy(row_hbm.at[wid, 0, pl.ds(off, BLK)], ridx_v)
            for g in range(BLK // _L16):
                r16 = ridx_v[pl.ds(g * _L16, _L16)]
                idx16 = iota16 + (b * BLK + g * _L16)
                m16 = jnp.where(idx16 < cnt_w, 1.0, 0.0)
                plsc.addupdate_scatter(deg_v, [izero16, r16], m16)
            return 0
        lax.fori_loop(0, nblk, blk_body, 0)
        pltpu.sync_copy(deg_v, deg_out.at[wid])


def _deg_kernel(row0, n0, row1, n1):
    mesh = plsc.VectorSubcoreMesh(core_axis_name="c", subcore_axis_name="s")
    return pl.kernel(
        _deg_body,
        out_type=(
            jax.ShapeDtypeStruct((NW, 1, N), jnp.float32),
            jax.ShapeDtypeStruct((NW, 1, N), jnp.float32),
        ),
        mesh=mesh,
        scratch_types=[
            pltpu.VMEM((1, N), jnp.float32),
            pltpu.VMEM((BLK,), jnp.int32),
            pltpu.VMEM((1, _L16), jnp.int32),
        ],
        compiler_params=pltpu.CompilerParams(needs_layout_passes=False),
    )(row0, n0, row1, n1)


def _edge_body(h1_hbm, ds_hbm, row_hbm, col_hbm, n_hbm,
               zeros_hbm, fout_hbm, s_hbm,
               accum, ds_v, ridxb, cidxb, cnt_v,
               hr0, hr1, hc0, hc1, rc0, rc1, cc0, cc1, s_v,
               gsr0, gsr1, gsc0, gsc1, gsr0b, gsr1b, gsc0b, gsc1b, ss0, ss1):
    cid = lax.axis_index("c")
    sid = lax.axis_index("s")
    wid = cid * NS + sid

    hr = (hr0, hr1)
    hc = (hc0, hc1)
    rcur = (rc0, rc1)
    ccur = (cc0, cc1)
    gsr = (gsr0, gsr1)
    gsc = (gsc0, gsc1)
    gsrb = (gsr0b, gsr1b)
    gscb = (gsc0b, gsc1b)
    ssem = (ss0, ss1)
    HS = SUB // 2

    @pl.when(sid == 0)
    def _():
        pltpu.sync_copy(zeros_hbm, accum)

    pltpu.sync_copy(ds_hbm, ds_v)
    pltpu.sync_copy(n_hbm.at[wid], cnt_v)
    cnt_w = jnp.max(cnt_v[0, :])
    nsub = lax.div(cnt_w + SUB - 1, SUB)
    nsr = lax.div(nsub + NSB - 1, NSB)
    plsc.subcore_barrier()

    iota16 = lax.iota(jnp.int32, _L16)
    z16 = jnp.zeros((_L16,), jnp.float32)
    e_base = [iota16 + g * _L16 for g in range(SUB // _L16)]

    def prep(par, jloc):
        off = jnp.minimum(jloc, NSB - 1) * SUB
        for g2 in range(SUB // _L16):
            sl = pl.ds(off + g2 * _L16, _L16)
            dsl = pl.ds(g2 * _L16, _L16)
            rcur[par][dsl] = jnp.minimum(jnp.maximum(ridxb[sl], 0), N - 1)
            ccur[par][dsl] = jnp.minimum(jnp.maximum(cidxb[sl], 0), N - 1)

    def _gparts(par):
        return (
            (h1_hbm.at[rcur[par].at[pl.ds(0, HS)]], hr[par].at[pl.ds(0, HS)], gsr[par]),
            (h1_hbm.at[rcur[par].at[pl.ds(HS, HS)]], hr[par].at[pl.ds(HS, HS)], gsrb[par]),
            (h1_hbm.at[ccur[par].at[pl.ds(0, HS)]], hc[par].at[pl.ds(0, HS)], gsc[par]),
            (h1_hbm.at[ccur[par].at[pl.ds(HS, HS)]], hc[par].at[pl.ds(HS, HS)], gscb[par]),
        )

    def gath(par):
        for s_, d_, m_ in _gparts(par):
            pltpu.async_copy(s_, d_, m_)

    def gath_wait(par):
        for s_, d_, m_ in _gparts(par):
            pltpu.make_async_copy(s_, d_, m_).wait()

    def scat(par):
        pltpu.async_copy(hr[par], accum.at[ccur[par]], ssem[par], add=True)

    def scat_wait(par):
        pltpu.make_async_copy(hr[par], accum.at[ccur[par]], ssem[par]).wait()

    eqs = [iota16 == l for l in range(_L16)]

    def compute(par, base_e, carry):
        hrv = hr[par]
        hcv = hc[par]

        def grp_body(g, carry2):
            s_num, s_den = carry2
            e0 = g * _L16
            dotv = z16
            qav = z16
            qbv = z16
            for le in range(_L16):
                e = e0 + le
                a = hrv[e, pl.ds(0, _L16)]
                b = hcv[e, pl.ds(0, _L16)]
                d_acc = a * b
                qa_acc = a * a
                qb_acc = b * b
                for f in range(1, H // _L16):
                    a = hrv[e, pl.ds(f * _L16, _L16)]
                    b = hcv[e, pl.ds(f * _L16, _L16)]
                    d_acc = d_acc + a * b
                    qa_acc = qa_acc + a * a
                    qb_acc = qb_acc + b * b
                dotv = jnp.where(eqs[le], jnp.sum(d_acc), dotv)
                qav = jnp.where(eqs[le], jnp.sum(qa_acc), qav)
                qbv = jnp.where(eqs[le], jnp.sum(qb_acc), qbv)

            r16 = rcur[par][pl.ds(e0, _L16)]
            c16 = ccur[par][pl.ds(e0, _L16)]
            idx16 = iota16 + (base_e + e0)
            m16 = jnp.where(idx16 < cnt_w, 1.0, 0.0)
            sr = plsc.load_gather(ds_v, [r16])
            sc = plsc.load_gather(ds_v, [c16])
            d_raw = jnp.maximum(qav + qbv - 2.0 * dotv, 0.0)
            d_scl = jnp.maximum(sr * sr * qav + sc * sc * qbv
                                - 2.0 * (sr * sc) * dotv, 0.0)
            g_raw = _tanh_pos_sc(_rsqrt_sc(d_raw + 1e-4))
            g_scl = _tanh_pos_sc(_rsqrt_sc(d_scl + 1e-4))
            srsc = sr * sc
            normv = g_scl * srsc * srsc * m16
            s_num = s_num + g_raw * m16
            s_den = s_den + m16

            for le in range(_L16):
                e = e0 + le
                nrm = jnp.sum(jnp.where(eqs[le], normv, 0.0))
                for f in range(H // _L16):
                    s2 = pl.ds(f * _L16, _L16)
                    hrv[e, s2] = hrv[e, s2] * nrm
            return (s_num, s_den)

        return lax.fori_loop(0, SUB // _L16, grp_body, carry)

    def round_body(sr_i, carry):
        s_num, s_den = carry
        soff = pl.multiple_of(sr_i * (NSB * SUB), NSB * SUB)
        pltpu.sync_copy(row_hbm.at[wid, 0, pl.ds(soff, NSB * SUB)], ridxb)
        pltpu.sync_copy(col_hbm.at[wid, 0, pl.ds(soff, NSB * SUB)], cidxb)
        nsb_r = jnp.minimum(NSB, nsub - sr_i * NSB)
        npairs = lax.div(nsb_r + 1, 2)
        base0 = sr_i * (NSB * SUB)

        prep(0, 0)
        gath(0)

        def pair_body(t, carry):
            s_num, s_den = carry
            j0 = 2 * t
            @pl.when(t > 0)
            def _():
                scat_wait(1)
            prep(1, j0 + 1)
            gath(1)
            gath_wait(0)
            s_num, s_den = compute(0, base0 + j0 * SUB, (s_num, s_den))
            scat(0)
            scat_wait(0)
            prep(0, j0 + 2)
            gath(0)
            gath_wait(1)
            s_num, s_den = compute(1, base0 + (j0 + 1) * SUB, (s_num, s_den))
            scat(1)
            return (s_num, s_den)

        s_num, s_den = lax.fori_loop(0, npairs, pair_body, (s_num, s_den))
        scat_wait(1)
        gath_wait(0)
        return (s_num, s_den)

    s_num, s_den = lax.fori_loop(0, nsr, round_body, (z16, z16))

    svec = jnp.where(iota16 == 0, jnp.sum(s_num),
                     jnp.where(iota16 == 1, jnp.sum(s_den), 0.0))
    s_v[0, :] = svec
    pltpu.sync_copy(s_v, s_hbm.at[wid])

    plsc.subcore_barrier()

    @pl.when(sid < NS - 1)
    def _():
        pltpu.sync_copy(accum.at[pl.ds(sid * RPT, RPT)],
                        fout_hbm.at[cid, pl.ds(sid * RPT, RPT)])

    @pl.when(sid == NS - 1)
    def _():
        pltpu.sync_copy(accum.at[pl.ds((NS - 1) * RPT, N - (NS - 1) * RPT)],
                        fout_hbm.at[cid, pl.ds((NS - 1) * RPT, N - (NS - 1) * RPT)])


def _edge_kernel(h1, ds_r, row_r, col_r, n_r, zeros):
    mesh = plsc.VectorSubcoreMesh(core_axis_name="c", subcore_axis_name="s")
    return pl.kernel(
        _edge_body,
        out_type=(
            jax.ShapeDtypeStruct((NC, N, H), jnp.float32),
            jax.ShapeDtypeStruct((NW, 1, _L16), jnp.float32),
        ),
        mesh=mesh,
        scratch_types=[
            pltpu.VMEM_SHARED((N, H), jnp.float32),
            pltpu.VMEM((N,), jnp.float32),
            pltpu.VMEM((NSB * SUB,), jnp.int32),
            pltpu.VMEM((NSB * SUB,), jnp.int32),
            pltpu.VMEM((1, _L16), jnp.int32),
            pltpu.VMEM((SUB, H), jnp.float32),
            pltpu.VMEM((SUB, H), jnp.float32),
            pltpu.VMEM((SUB, H), jnp.float32),
            pltpu.VMEM((SUB, H), jnp.float32),
            pltpu.VMEM((SUB,), jnp.int32),
            pltpu.VMEM((SUB,), jnp.int32),
            pltpu.VMEM((SUB,), jnp.int32),
            pltpu.VMEM((SUB,), jnp.int32),
            pltpu.VMEM((1, _L16), jnp.float32),
            pltpu.SemaphoreType.DMA,
            pltpu.SemaphoreType.DMA,
            pltpu.SemaphoreType.DMA,
            pltpu.SemaphoreType.DMA,
            pltpu.SemaphoreType.DMA,
            pltpu.SemaphoreType.DMA,
            pltpu.SemaphoreType.DMA,
            pltpu.SemaphoreType.DMA,
            pltpu.SemaphoreType.DMA,
            pltpu.SemaphoreType.DMA,
        ],
        compiler_params=pltpu.CompilerParams(needs_layout_passes=False),
    )(h1, ds_r, row_r, col_r, n_r, zeros)


MB = 1000
MG = N // MB


def _mm1_body(h_ref, w_ref, b_ref, h1_ref):
    h1 = jnp.dot(h_ref[...], w_ref[...], preferred_element_type=jnp.float32)
    h1_ref[...] = jnp.maximum(h1 + b_ref[...], 0.0)


def _mm1(h, w1t, b1):
    return pl.pallas_call(
        _mm1_body,
        grid=(MG,),
        in_specs=[
            pl.BlockSpec((MB, F_IN), lambda i: (i, 0)),
            pl.BlockSpec((F_IN, H), lambda i: (0, 0)),
            pl.BlockSpec((1, H), lambda i: (0, 0)),
        ],
        out_specs=pl.BlockSpec((MB, H), lambda i: (i, 0)),
        out_shape=jax.ShapeDtypeStruct((N, H), jnp.float32),
    )(h, w1t, b1)


def _prep_body(p0_ref, p1_ref, ds_ref):
    d0 = jnp.sum(p0_ref[...], axis=(0, 1))
    d1 = jnp.sum(p1_ref[...], axis=(0, 1))
    deg = jnp.maximum(jnp.stack([d0, d1]), 1.0)
    ds_ref[...] = lax.sqrt(lax.rsqrt(deg))


def _prep(deg0, deg1):
    return pl.pallas_call(
        _prep_body,
        out_shape=jax.ShapeDtypeStruct((R, N), jnp.float32),
    )(deg0, deg1)


def _combine_body(raw_ref, f0_ref, f1_ref, s0_ref, s1_ref, h1_ref):
    s0 = s0_ref[...]
    s1 = s1_ref[...]
    sr0 = jnp.sum(s0[:, 0, 0]) / jnp.sum(s0[:, 0, 1])
    sr1 = jnp.sum(s1[:, 0, 0]) / jnp.sum(s1[:, 0, 1])
    l1tr = jnp.abs(sr0) + jnp.abs(sr1)
    fi = l1tr + 2.0 * LAMDA2 / LAMDA1
    cl = 2.0 * LAMDA2 / LAMDA1
    u0 = jnp.float32(0.5)
    u1 = jnp.float32(0.5)
    for it in range(ITERS):
        t_t = np.float32(np.sqrt(2.0 * np.log(R) / (it + 1.0))) / fi
        u0t = u0 * jnp.exp(-t_t * (cl * u0 + sr0))
        u1t = u1 * jnp.exp(-t_t * (cl * u1 + sr1))
        ssum = u0t + u1t
        u0 = u0t / ssum
        u1 = u1t / ssum
    f_sum = u0 * (f0_ref[0] + f0_ref[1]) + u1 * (f1_ref[0] + f1_ref[1])
    h1_ref[...] = ALPHA * raw_ref[...] + (1.0 - ALPHA) * f_sum


def _combine(raw, f0, f1, s0, s1):
    return pl.pallas_call(
        _combine_body,
        grid=(MG,),
        in_specs=[
            pl.BlockSpec((MB, H), lambda i: (i, 0)),
            pl.BlockSpec((NC, MB, H), lambda i: (0, i, 0)),
            pl.BlockSpec((NC, MB, H), lambda i: (0, i, 0)),
            pl.BlockSpec((NW, 1, _L16), lambda i: (0, 0, 0)),
            pl.BlockSpec((NW, 1, _L16), lambda i: (0, 0, 0)),
        ],
        out_specs=pl.BlockSpec((MB, H), lambda i: (i, 0)),
        out_shape=jax.ShapeDtypeStruct((N, H), jnp.float32),
    )(raw, f0, f1, s0, s1)


def _final_body(h1_ref, w_ref, b_ref, lp_ref, lg_ref):
    logits = jnp.dot(h1_ref[...], w_ref[...], preferred_element_type=jnp.float32)
    logits = logits + b_ref[...]
    m = jnp.max(logits, axis=1, keepdims=True)
    lse = m + jnp.log(jnp.sum(jnp.exp(logits - m), axis=1, keepdims=True))
    lg_ref[...] = logits
    lp_ref[...] = logits - lse


def _final(h1, w2t, b2):
    return pl.pallas_call(
        _final_body,
        grid=(MG,),
        in_specs=[
            pl.BlockSpec((MB, H), lambda i: (i, 0)),
            pl.BlockSpec((H, C), lambda i: (0, 0)),
            pl.BlockSpec((1, C), lambda i: (0, 0)),
        ],
        out_specs=(
            pl.BlockSpec((MB, C), lambda i: (i, 0)),
            pl.BlockSpec((MB, C), lambda i: (i, 0)),
        ),
        out_shape=(
            jax.ShapeDtypeStruct((N, C), jnp.float32),
            jax.ShapeDtypeStruct((N, C), jnp.float32),
        ),
    )(h1, w2t, b2)


def kernel(h, adj_list, labels, W1, b1, W2, b2):
    r0, c0, n0, r1, c1, n1 = _extract(adj_list)
    rows, cols, cnts = (r0, r1), (c0, c1), (n0, n1)

    h1 = _mm1(h, W1.T, b1.reshape(1, H))
    raw = h1
    deg0, deg1 = _deg_kernel(rows[0], cnts[0], rows[1], cnts[1])
    ds = _prep(deg0, deg1)
    zeros = jnp.zeros((N, H), jnp.float32)

    for _ in range(LAYER_NUM):
        f_list, s_list = [], []
        for r in range(R):
            fpart, spart = _edge_kernel(h1, ds[r],
                                        rows[r], cols[r], cnts[r], zeros)
            f_list.append(fpart)
            s_list.append(spart)
        h1 = _combine(raw, f_list[0], f_list[1], s_list[0], s_list[1])

    return _final(h1, W2.T, b2.reshape(1, C))
